# Initial kernel scaffold; baseline (speedup 1.0000x reference)
import jax
import jax.numpy as jnp
from jax.experimental import pallas as pl


def kernel(z, edge_index, distances, params):
    raise NotImplementedError("write your pallas kernel here")


# speedup vs baseline: 2.0308x; 2.0308x over previous
import functools

import jax
import jax.numpy as jnp
from jax import lax
from jax.experimental import pallas as pl
from jax.experimental.pallas import tpu as pltpu
from jax.experimental.pallas import tpu_sc as plsc

N = 10000
E = 160000
C = 64
G = 150

NC = 2
NS = 16
NW = NC * NS
EP = 163840
EPW = EP // NW
CH = 1024
KJ = CH // 128
NCHUNK = EPW // CH
IRPW = EPW // 128
NP = 10240
NPS = NP // NS

EBLK = 2048
NBLK = 2000
GP = 256

_LOG2 = 0.6931471805599453
_GCOEF = 0.1 ** 0.5


def _ssp(x):
    return jnp.maximum(x, 0.0) + jnp.log1p(jnp.exp(-jnp.abs(x))) - _LOG2


def _dot(a, b):
    return jnp.dot(a, b, preferred_element_type=jnp.float32)


def _embed_body(z_ref, emb_ref, h_ref):
    zb = z_ref[...]
    oh = (zb == lax.broadcasted_iota(jnp.int32, (NBLK, 100), 1)).astype(jnp.float32)
    h_ref[...] = _dot(oh, emb_ref[...])


def _embed(z2d, emb):
    return pl.pallas_call(
        _embed_body,
        grid=(N // NBLK,),
        in_specs=[
            pl.BlockSpec((NBLK, 1), lambda i: (i, 0)),
            pl.BlockSpec((100, C), lambda i: (0, 0)),
        ],
        out_specs=pl.BlockSpec((NBLK, C), lambda i: (i, 0)),
        out_shape=jax.ShapeDtypeStruct((N, C), jnp.float32),
    )(z2d, emb)


def _edge_mlp_body(first, hs_ref, hd_ref, ea_ref, w1s_ref, w1d_ref, w1e_ref,
                   b1_ref, w2_ref, b2_ref, fw_ref, fb_ref, ea_out_ref, msg_ref):
    hs = hs_ref[...]
    hd = hd_ref[...]
    if first:
        d = ea_ref[...]
        kk = lax.broadcasted_iota(jnp.int32, (EBLK, GP), 1).astype(jnp.float32)
        diff = d - 0.1 * kk
        ea_in = jnp.exp(-_GCOEF * diff * diff)
    else:
        ea_in = ea_ref[...]
    pre = (_dot(hs, w1s_ref[...]) + _dot(hd, w1d_ref[...])
           + _dot(ea_in, w1e_ref[...]) + b1_ref[...])
    ea = _dot(_ssp(pre), w2_ref[...]) + b2_ref[...]
    hh = _dot(hs, fw_ref[...]) + fb_ref[...]
    rid = pl.program_id(0) * EBLK + lax.broadcasted_iota(jnp.int32, (EBLK, 1), 0)
    valid = (rid < E).astype(jnp.float32)
    ea_ref_out = ea * valid
    ea_out_ref[...] = ea_ref_out
    msg_ref[...] = hh * ea_ref_out


def _edge_mlp(first, hs, hd, ea_or_d, w1s, w1d, w1e, b1, w2, b2, fw, fb):
    ein = ea_or_d.shape[1]
    win = w1e.shape[0]
    return pl.pallas_call(
        functools.partial(_edge_mlp_body, first),
        grid=(EP // EBLK,),
        in_specs=[
            pl.BlockSpec((EBLK, C), lambda i: (i, 0)),
            pl.BlockSpec((EBLK, C), lambda i: (i, 0)),
            pl.BlockSpec((EBLK, ein), lambda i: (i, 0)),
            pl.BlockSpec((C, 2 * C), lambda i: (0, 0)),
            pl.BlockSpec((C, 2 * C), lambda i: (0, 0)),
            pl.BlockSpec((win, 2 * C), lambda i: (0, 0)),
            pl.BlockSpec((1, 2 * C), lambda i: (0, 0)),
            pl.BlockSpec((2 * C, C), lambda i: (0, 0)),
            pl.BlockSpec((1, C), lambda i: (0, 0)),
            pl.BlockSpec((C, C), lambda i: (0, 0)),
            pl.BlockSpec((1, C), lambda i: (0, 0)),
        ],
        out_specs=[
            pl.BlockSpec((EBLK, C), lambda i: (i, 0)),
            pl.BlockSpec((EBLK, C), lambda i: (i, 0)),
        ],
        out_shape=[
            jax.ShapeDtypeStruct((EP, C), jnp.float32),
            jax.ShapeDtypeStruct((EP, C), jnp.float32),
        ],
    )(hs, hd, ea_or_d, w1s, w1d, w1e, b1, w2, b2, fw, fb)


def _node_body(h_ref, m_ref, w1_ref, b1_ref, w2_ref, b2_ref, h_out_ref):
    m = m_ref[0] + m_ref[1]
    t = _ssp(_dot(m, w1_ref[...]) + b1_ref[...])
    h_out_ref[...] = h_ref[...] + _dot(t, w2_ref[...]) + b2_ref[...]


def _node(h, mout, w1, b1, w2, b2):
    return pl.pallas_call(
        _node_body,
        grid=(N // NBLK,),
        in_specs=[
            pl.BlockSpec((NBLK, C), lambda i: (i, 0)),
            pl.BlockSpec((NC, NBLK, C), lambda i: (0, i, 0)),
            pl.BlockSpec((C, C), lambda i: (0, 0)),
            pl.BlockSpec((1, C), lambda i: (0, 0)),
            pl.BlockSpec((C, C), lambda i: (0, 0)),
            pl.BlockSpec((1, C), lambda i: (0, 0)),
        ],
        out_specs=pl.BlockSpec((NBLK, C), lambda i: (i, 0)),
        out_shape=jax.ShapeDtypeStruct((N, C), jnp.float32),
    )(h, mout, w1, b1, w2, b2)


def _node_final_body(h_ref, m_ref, w1_ref, b1_ref, w2_ref, b2_ref,
                     hw1_ref, hb1_ref, hw2_ref, hb2_ref, out_ref):
    m = m_ref[0] + m_ref[1]
    t = _ssp(_dot(m, w1_ref[...]) + b1_ref[...])
    h = h_ref[...] + _dot(t, w2_ref[...]) + b2_ref[...]
    t2 = _ssp(_dot(h, hw1_ref[...]) + hb1_ref[...])
    hv = _dot(t2, hw2_ref[...]) + hb2_ref[...]
    part = jnp.sum(hv)

    @pl.when(pl.program_id(0) == 0)
    def _():
        out_ref[...] = jnp.zeros_like(out_ref)

    out_ref[...] += part[None, None]


def _node_final(h, mout, w1, b1, w2, b2, hw1, hb1, hw2, hb2):
    return pl.pallas_call(
        _node_final_body,
        grid=(N // NBLK,),
        in_specs=[
            pl.BlockSpec((NBLK, C), lambda i: (i, 0)),
            pl.BlockSpec((NC, NBLK, C), lambda i: (0, i, 0)),
            pl.BlockSpec((C, C), lambda i: (0, 0)),
            pl.BlockSpec((1, C), lambda i: (0, 0)),
            pl.BlockSpec((C, C), lambda i: (0, 0)),
            pl.BlockSpec((1, C), lambda i: (0, 0)),
            pl.BlockSpec((C, C // 2), lambda i: (0, 0)),
            pl.BlockSpec((1, C // 2), lambda i: (0, 0)),
            pl.BlockSpec((C // 2, 1), lambda i: (0, 0)),
            pl.BlockSpec((1, 1), lambda i: (0, 0)),
        ],
        out_specs=pl.BlockSpec((1, 1), lambda i: (0, 0)),
        out_shape=jax.ShapeDtypeStruct((1, 1), jnp.float32),
    )(h, mout, w1, b1, w2, b2, hw1, hb1, hw2, hb2)


@functools.cache
def _sc_mesh():
    return plsc.VectorSubcoreMesh(core_axis_name="c", subcore_axis_name="s",
                                  num_cores=NC, num_subcores=NS)


def _gather_body(h_hbm, src_hbm, dst_hbm, hs_out, hd_out,
                 idx, rows, sem):
    cid = lax.axis_index("c")
    sid = lax.axis_index("s")
    wid = sid * NC + cid

    def one_table(idx_hbm, out_hbm):
        def chunk(i, carry):
            r0 = wid * IRPW + i * KJ
            e0 = wid * EPW + i * CH
            pltpu.sync_copy(idx_hbm.at[pl.ds(r0, KJ)], idx)
            cps = []
            for j in range(KJ):
                cps.append(pltpu.async_copy(
                    h_hbm.at[idx.at[j]], rows.at[pl.ds(j * 128, 128)], sem))
            for cp in cps:
                cp.wait()
            pltpu.sync_copy(rows, out_hbm.at[pl.ds(e0, CH)])
            return carry

        lax.fori_loop(0, NCHUNK, chunk, 0)

    one_table(src_hbm, hs_out)
    one_table(dst_hbm, hd_out)


def _gather(h, src2d, dst2d):
    return pl.kernel(
        _gather_body,
        out_type=[
            jax.ShapeDtypeStruct((EP, C), jnp.float32),
            jax.ShapeDtypeStruct((EP, C), jnp.float32),
        ],
        mesh=_sc_mesh(),
        compiler_params=pltpu.CompilerParams(use_tc_tiling_on_sc=False),
        scratch_types=[
            pltpu.VMEM((KJ, 128), jnp.int32),
            pltpu.VMEM((CH, C), jnp.float32),
            pltpu.SemaphoreType.DMA,
        ],
    )(h, src2d, dst2d)


def _scatter_body(msg_hbm, dst_hbm, zero_hbm, mout_hbm, macc, mbuf, idx_d, sem):
    cid = lax.axis_index("c")
    sid = lax.axis_index("s")
    wid = sid * NC + cid

    pltpu.sync_copy(zero_hbm.at[pl.ds(sid * NPS, NPS)],
                    macc.at[pl.ds(sid * NPS, NPS)])
    plsc.subcore_barrier()

    def chunk(i, carry):
        r0 = wid * IRPW + i * KJ
        e0 = wid * EPW + i * CH
        pltpu.sync_copy(dst_hbm.at[pl.ds(r0, KJ)], idx_d)
        pltpu.sync_copy(msg_hbm.at[pl.ds(e0, CH)], mbuf)
        for j in range(KJ):
            pltpu.sync_copy(mbuf.at[pl.ds(j * 128, 128)],
                            macc.at[idx_d.at[j]], add=True)
        return carry

    lax.fori_loop(0, NCHUNK, chunk, 0)
    plsc.subcore_barrier()
    pltpu.sync_copy(macc.at[pl.ds(sid * NPS, NPS)],
                    mout_hbm.at[cid, pl.ds(sid * NPS, NPS)])


def _scatter(msg, dst2d, zeros_np):
    return pl.kernel(
        _scatter_body,
        out_type=jax.ShapeDtypeStruct((NC, NP, C), jnp.float32),
        mesh=_sc_mesh(),
        compiler_params=pltpu.CompilerParams(use_tc_tiling_on_sc=False),
        scratch_types=[
            pltpu.VMEM_SHARED((NP, C), jnp.float32),
            pltpu.VMEM((CH, C), jnp.float32),
            pltpu.VMEM((KJ, 128), jnp.int32),
            pltpu.SemaphoreType.DMA,
        ],
    )(msg, dst2d, zeros_np)


def kernel(z, edge_index, distances, params):
    src = edge_index[0].astype(jnp.int32)
    dst = edge_index[1].astype(jnp.int32)
    src2d = jnp.pad(src, (0, EP - E)).reshape(EP // 128, 128)
    dst2d = jnp.pad(dst, (0, EP - E)).reshape(EP // 128, 128)
    dpad = jnp.pad(distances.astype(jnp.float32), (0, EP - E)).reshape(EP, 1)
    zeros_np = jnp.zeros((NP, C), jnp.float32)

    h = _embed(z.reshape(N, 1).astype(jnp.int32), params["emb"])

    ea = dpad
    out = None
    for i in range(3):
        eu = params["eu"][i]
        it = params["it"][i]
        w1 = eu["W1"]
        w1s = w1[:C]
        w1d = w1[C:2 * C]
        w1e = w1[2 * C:]
        if i == 0:
            w1e = jnp.zeros((GP, 2 * C), jnp.float32).at[:G].set(w1e)
        b1 = eu["b1"].reshape(1, 2 * C)
        b2 = eu["b2"].reshape(1, C)
        fw = it["fW"]
        fb = it["fb"].reshape(1, C)

        hs, hd = _gather(h, src2d, dst2d)
        ea, msg = _edge_mlp(i == 0, hs, hd, ea, w1s, w1d, w1e, b1,
                            eu["W2"], b2, fw, fb)
        mout = _scatter(msg, dst2d, zeros_np)

        m1w1 = it["m1W1"]
        m1b1 = it["m1b1"].reshape(1, C)
        m1w2 = it["m1W2"]
        m1b2 = it["m1b2"].reshape(1, C)
        if i < 2:
            h = _node(h, mout, m1w1, m1b1, m1w2, m1b2)
        else:
            hd_p = params["head"]
            out = _node_final(h, mout, m1w1, m1b1, m1w2, m1b2,
                              hd_p["W1"], hd_p["b1"].reshape(1, C // 2),
                              hd_p["W2"], hd_p["b2"].reshape(1, 1))
    return out


# speedup vs baseline: 2.1397x; 1.0536x over previous
import functools

import jax
import jax.numpy as jnp
from jax import lax
from jax.experimental import pallas as pl
from jax.experimental.pallas import tpu as pltpu
from jax.experimental.pallas import tpu_sc as plsc

N = 10000
E = 160000
C = 64
G = 150

NC = 2
NS = 16
NW = NC * NS
EP = 163840
EPW = EP // NW
CH = 640
KJ = CH // 128
NCHUNK = EPW // CH
IRPW = EPW // 128
NP = 10240
NPS = NP // NS

EBLK = 2048
NBLK = 2000
GP = 256

_LOG2 = 0.6931471805599453
_GCOEF = 0.1 ** 0.5


def _ssp(x):
    return jnp.maximum(x, 0.0) + jnp.log1p(jnp.exp(-jnp.abs(x))) - _LOG2


def _dot(a, b):
    return jnp.dot(a, b, preferred_element_type=jnp.float32)


def _embed_body(z_ref, emb_ref, h_ref):
    zb = z_ref[...]
    oh = (zb == lax.broadcasted_iota(jnp.int32, (NBLK, 100), 1)).astype(jnp.float32)
    h_ref[...] = _dot(oh, emb_ref[...])


def _embed(z2d, emb):
    return pl.pallas_call(
        _embed_body,
        grid=(N // NBLK,),
        in_specs=[
            pl.BlockSpec((NBLK, 1), lambda i: (i, 0)),
            pl.BlockSpec((100, C), lambda i: (0, 0)),
        ],
        out_specs=pl.BlockSpec((NBLK, C), lambda i: (i, 0)),
        out_shape=jax.ShapeDtypeStruct((N, C), jnp.float32),
    )(z2d, emb)


def _edge_mlp_body(first, hs_ref, hd_ref, ea_ref, w1s_ref, w1d_ref, w1e_ref,
                   b1_ref, w2_ref, b2_ref, fw_ref, fb_ref, ea_out_ref, msg_ref):
    hs = hs_ref[...]
    hd = hd_ref[...]
    if first:
        d = ea_ref[...]
        kk = lax.broadcasted_iota(jnp.int32, (EBLK, GP), 1).astype(jnp.float32)
        diff = d - 0.1 * kk
        ea_in = jnp.exp(-_GCOEF * diff * diff)
    else:
        ea_in = ea_ref[...]
    pre = (_dot(hs, w1s_ref[...]) + _dot(hd, w1d_ref[...])
           + _dot(ea_in, w1e_ref[...]) + b1_ref[...])
    ea = _dot(_ssp(pre), w2_ref[...]) + b2_ref[...]
    hh = _dot(hs, fw_ref[...]) + fb_ref[...]
    rid = pl.program_id(0) * EBLK + lax.broadcasted_iota(jnp.int32, (EBLK, 1), 0)
    valid = (rid < E).astype(jnp.float32)
    ea_ref_out = ea * valid
    ea_out_ref[...] = ea_ref_out
    msg_ref[...] = hh * ea_ref_out


def _edge_mlp(first, hs, hd, ea_or_d, w1s, w1d, w1e, b1, w2, b2, fw, fb):
    ein = ea_or_d.shape[1]
    win = w1e.shape[0]
    return pl.pallas_call(
        functools.partial(_edge_mlp_body, first),
        grid=(EP // EBLK,),
        in_specs=[
            pl.BlockSpec((EBLK, C), lambda i: (i, 0)),
            pl.BlockSpec((EBLK, C), lambda i: (i, 0)),
            pl.BlockSpec((EBLK, ein), lambda i: (i, 0)),
            pl.BlockSpec((C, 2 * C), lambda i: (0, 0)),
            pl.BlockSpec((C, 2 * C), lambda i: (0, 0)),
            pl.BlockSpec((win, 2 * C), lambda i: (0, 0)),
            pl.BlockSpec((1, 2 * C), lambda i: (0, 0)),
            pl.BlockSpec((2 * C, C), lambda i: (0, 0)),
            pl.BlockSpec((1, C), lambda i: (0, 0)),
            pl.BlockSpec((C, C), lambda i: (0, 0)),
            pl.BlockSpec((1, C), lambda i: (0, 0)),
        ],
        out_specs=[
            pl.BlockSpec((EBLK, C), lambda i: (i, 0)),
            pl.BlockSpec((EBLK, C), lambda i: (i, 0)),
        ],
        out_shape=[
            jax.ShapeDtypeStruct((EP, C), jnp.float32),
            jax.ShapeDtypeStruct((EP, C), jnp.float32),
        ],
    )(hs, hd, ea_or_d, w1s, w1d, w1e, b1, w2, b2, fw, fb)


def _node_body(h_ref, m_ref, w1_ref, b1_ref, w2_ref, b2_ref, h_out_ref):
    m = m_ref[0] + m_ref[1]
    t = _ssp(_dot(m, w1_ref[...]) + b1_ref[...])
    h_out_ref[...] = h_ref[...] + _dot(t, w2_ref[...]) + b2_ref[...]


def _node(h, mout, w1, b1, w2, b2):
    return pl.pallas_call(
        _node_body,
        grid=(N // NBLK,),
        in_specs=[
            pl.BlockSpec((NBLK, C), lambda i: (i, 0)),
            pl.BlockSpec((NC, NBLK, C), lambda i: (0, i, 0)),
            pl.BlockSpec((C, C), lambda i: (0, 0)),
            pl.BlockSpec((1, C), lambda i: (0, 0)),
            pl.BlockSpec((C, C), lambda i: (0, 0)),
            pl.BlockSpec((1, C), lambda i: (0, 0)),
        ],
        out_specs=pl.BlockSpec((NBLK, C), lambda i: (i, 0)),
        out_shape=jax.ShapeDtypeStruct((N, C), jnp.float32),
    )(h, mout, w1, b1, w2, b2)


def _node_final_body(h_ref, m_ref, w1_ref, b1_ref, w2_ref, b2_ref,
                     hw1_ref, hb1_ref, hw2_ref, hb2_ref, out_ref):
    m = m_ref[0] + m_ref[1]
    t = _ssp(_dot(m, w1_ref[...]) + b1_ref[...])
    h = h_ref[...] + _dot(t, w2_ref[...]) + b2_ref[...]
    t2 = _ssp(_dot(h, hw1_ref[...]) + hb1_ref[...])
    hv = _dot(t2, hw2_ref[...]) + hb2_ref[...]
    part = jnp.sum(hv)

    @pl.when(pl.program_id(0) == 0)
    def _():
        out_ref[...] = jnp.zeros_like(out_ref)

    out_ref[...] += part[None, None]


def _node_final(h, mout, w1, b1, w2, b2, hw1, hb1, hw2, hb2):
    return pl.pallas_call(
        _node_final_body,
        grid=(N // NBLK,),
        in_specs=[
            pl.BlockSpec((NBLK, C), lambda i: (i, 0)),
            pl.BlockSpec((NC, NBLK, C), lambda i: (0, i, 0)),
            pl.BlockSpec((C, C), lambda i: (0, 0)),
            pl.BlockSpec((1, C), lambda i: (0, 0)),
            pl.BlockSpec((C, C), lambda i: (0, 0)),
            pl.BlockSpec((1, C), lambda i: (0, 0)),
            pl.BlockSpec((C, C // 2), lambda i: (0, 0)),
            pl.BlockSpec((1, C // 2), lambda i: (0, 0)),
            pl.BlockSpec((C // 2, 1), lambda i: (0, 0)),
            pl.BlockSpec((1, 1), lambda i: (0, 0)),
        ],
        out_specs=pl.BlockSpec((1, 1), lambda i: (0, 0)),
        out_shape=jax.ShapeDtypeStruct((1, 1), jnp.float32),
    )(h, mout, w1, b1, w2, b2, hw1, hb1, hw2, hb2)


@functools.cache
def _sc_mesh():
    return plsc.VectorSubcoreMesh(core_axis_name="c", subcore_axis_name="s",
                                  num_cores=NC, num_subcores=NS)


def _gather_body(h_hbm, src_hbm, dst_hbm, hs_out, hd_out,
                 idx, rows0, rows1, semg0, semg1, semw0, semw1):
    cid = lax.axis_index("c")
    sid = lax.axis_index("s")
    wid = sid * NC + cid

    cpi0 = pltpu.async_copy(src_hbm.at[pl.ds(wid * IRPW, IRPW)],
                            idx.at[pl.ds(0, IRPW)], semg0)
    cpi1 = pltpu.async_copy(dst_hbm.at[pl.ds(wid * IRPW, IRPW)],
                            idx.at[pl.ds(IRPW, IRPW)], semg1)
    cpi0.wait()
    cpi1.wait()

    rows = [rows0, rows1]
    semg = [semg0, semg1]
    semw = [semw0, semw1]
    outs = [hs_out, hd_out]
    steps = 2 * NCHUNK

    def issue_gathers(s, buf, sem):
        t, i = divmod(s, NCHUNK)
        cps = []
        for j in range(KJ):
            cps.append(pltpu.async_copy(
                h_hbm.at[idx.at[t * IRPW + i * KJ + j]],
                buf.at[pl.ds(j * 128, 128)], sem))
        return cps

    def issue_writeout(s, buf, sem):
        t, i = divmod(s, NCHUNK)
        return pltpu.async_copy(buf, outs[t].at[pl.ds(wid * EPW + i * CH, CH)],
                                sem)

    wc = [None, None]
    g = [issue_gathers(0, rows0, semg0), None]
    for s in range(steps):
        b = s % 2
        nb = 1 - b
        if s + 1 < steps:
            if wc[nb] is not None:
                wc[nb].wait()
            g[nb] = issue_gathers(s + 1, rows[nb], semg[nb])
        for cp in g[b]:
            cp.wait()
        wc[b] = issue_writeout(s, rows[b], semw[b])
    wc[0].wait()
    wc[1].wait()


def _gather(h, src2d, dst2d):
    return pl.kernel(
        _gather_body,
        out_type=[
            jax.ShapeDtypeStruct((EP, C), jnp.float32),
            jax.ShapeDtypeStruct((EP, C), jnp.float32),
        ],
        mesh=_sc_mesh(),
        compiler_params=pltpu.CompilerParams(use_tc_tiling_on_sc=False),
        scratch_types=[
            pltpu.VMEM((2 * IRPW, 128), jnp.int32),
            pltpu.VMEM((CH, C), jnp.float32),
            pltpu.VMEM((CH, C), jnp.float32),
            pltpu.SemaphoreType.DMA,
            pltpu.SemaphoreType.DMA,
            pltpu.SemaphoreType.DMA,
            pltpu.SemaphoreType.DMA,
        ],
    )(h, src2d, dst2d)


def _scatter_body(msg_hbm, dst_hbm, zero_hbm, mout_hbm, macc, mbuf0, mbuf1,
                  idx, semm0, semm1, sema0, sema1):
    cid = lax.axis_index("c")
    sid = lax.axis_index("s")
    wid = sid * NC + cid

    cpi = pltpu.async_copy(dst_hbm.at[pl.ds(wid * IRPW, IRPW)], idx, semm0)
    cpm0 = pltpu.async_copy(msg_hbm.at[pl.ds(wid * EPW, CH)], mbuf0, semm1)
    pltpu.sync_copy(zero_hbm.at[pl.ds(sid * NPS, NPS)],
                    macc.at[pl.ds(sid * NPS, NPS)])
    cpi.wait()
    plsc.subcore_barrier()

    mbuf = [mbuf0, mbuf1]
    semm = [semm0, semm1]
    sema = [sema0, sema1]

    def issue_adds(i, buf, sem):
        cps = []
        for j in range(KJ):
            cps.append(pltpu.async_copy(
                buf.at[pl.ds(j * 128, 128)], macc.at[idx.at[i * KJ + j]],
                sem, add=True))
        return cps

    mc = [cpm0, None]
    ac = [None, None]
    for i in range(NCHUNK):
        b = i % 2
        nb = 1 - b
        if i + 1 < NCHUNK:
            if ac[nb] is not None:
                for cp in ac[nb]:
                    cp.wait()
                ac[nb] = None
            mc[nb] = pltpu.async_copy(
                msg_hbm.at[pl.ds(wid * EPW + (i + 1) * CH, CH)],
                mbuf[nb], semm[nb])
        mc[b].wait()
        ac[b] = issue_adds(i, mbuf[b], sema[b])
    for cps in ac:
        if cps is not None:
            for cp in cps:
                cp.wait()
    plsc.subcore_barrier()
    pltpu.sync_copy(macc.at[pl.ds(sid * NPS, NPS)],
                    mout_hbm.at[cid, pl.ds(sid * NPS, NPS)])


def _scatter(msg, dst2d, zeros_np):
    return pl.kernel(
        _scatter_body,
        out_type=jax.ShapeDtypeStruct((NC, NP, C), jnp.float32),
        mesh=_sc_mesh(),
        compiler_params=pltpu.CompilerParams(use_tc_tiling_on_sc=False),
        scratch_types=[
            pltpu.VMEM_SHARED((NP, C), jnp.float32),
            pltpu.VMEM((CH, C), jnp.float32),
            pltpu.VMEM((CH, C), jnp.float32),
            pltpu.VMEM((IRPW, 128), jnp.int32),
            pltpu.SemaphoreType.DMA,
            pltpu.SemaphoreType.DMA,
            pltpu.SemaphoreType.DMA,
            pltpu.SemaphoreType.DMA,
        ],
    )(msg, dst2d, zeros_np)


def kernel(z, edge_index, distances, params):
    src = edge_index[0].astype(jnp.int32)
    dst = edge_index[1].astype(jnp.int32)
    src2d = jnp.pad(src, (0, EP - E)).reshape(EP // 128, 128)
    dst2d = jnp.pad(dst, (0, EP - E)).reshape(EP // 128, 128)
    dpad = jnp.pad(distances.astype(jnp.float32), (0, EP - E)).reshape(EP, 1)
    zeros_np = jnp.zeros((NP, C), jnp.float32)

    h = _embed(z.reshape(N, 1).astype(jnp.int32), params["emb"])

    ea = dpad
    out = None
    for i in range(3):
        eu = params["eu"][i]
        it = params["it"][i]
        w1 = eu["W1"]
        w1s = w1[:C]
        w1d = w1[C:2 * C]
        w1e = w1[2 * C:]
        if i == 0:
            w1e = jnp.zeros((GP, 2 * C), jnp.float32).at[:G].set(w1e)
        b1 = eu["b1"].reshape(1, 2 * C)
        b2 = eu["b2"].reshape(1, C)
        fw = it["fW"]
        fb = it["fb"].reshape(1, C)

        hs, hd = _gather(h, src2d, dst2d)
        ea, msg = _edge_mlp(i == 0, hs, hd, ea, w1s, w1d, w1e, b1,
                            eu["W2"], b2, fw, fb)
        mout = _scatter(msg, dst2d, zeros_np)

        m1w1 = it["m1W1"]
        m1b1 = it["m1b1"].reshape(1, C)
        m1w2 = it["m1W2"]
        m1b2 = it["m1b2"].reshape(1, C)
        if i < 2:
            h = _node(h, mout, m1w1, m1b1, m1w2, m1b2)
        else:
            hd_p = params["head"]
            out = _node_final(h, mout, m1w1, m1b1, m1w2, m1b2,
                              hd_p["W1"], hd_p["b1"].reshape(1, C // 2),
                              hd_p["W2"], hd_p["b2"].reshape(1, 1))
    return out


# speedup vs baseline: 3.2004x; 1.4957x over previous
import functools

import jax
import jax.numpy as jnp
from jax import lax
from jax.experimental import pallas as pl
from jax.experimental.pallas import tpu as pltpu
from jax.experimental.pallas import tpu_sc as plsc

N = 10000
E = 160000
C = 64
G = 150

NC = 2
NS = 16
NW = NC * NS
EP = 163840
EPW = EP // NW
CH = 512
KJ = CH // 128
NCHUNK = EPW // CH
IRPW = EPW // 128
NP = 10240
NPS = NP // NS

EBLK = 2048
NBLK = 2000
GP = 256

_LOG2 = 0.6931471805599453
_GCOEF = 0.1 ** 0.5


def _ssp(x):
    return jnp.maximum(x, 0.0) + jnp.log1p(jnp.exp(-jnp.abs(x))) - _LOG2


def _dot(a, b):
    return jnp.dot(a, b, preferred_element_type=jnp.float32)


def _embed_body(z_ref, emb_ref, h_ref):
    zb = z_ref[...]
    oh = (zb == lax.broadcasted_iota(jnp.int32, (NBLK, 100), 1)).astype(jnp.float32)
    h_ref[...] = _dot(oh, emb_ref[...])


def _embed(z2d, emb):
    return pl.pallas_call(
        _embed_body,
        grid=(N // NBLK,),
        in_specs=[
            pl.BlockSpec((NBLK, 1), lambda i: (i, 0)),
            pl.BlockSpec((100, C), lambda i: (0, 0)),
        ],
        out_specs=pl.BlockSpec((NBLK, C), lambda i: (i, 0)),
        out_shape=jax.ShapeDtypeStruct((N, C), jnp.float32),
    )(z2d, emb)


def _edge_mlp_body(first, hs_ref, hd_ref, ea_ref, w1s_ref, w1d_ref, w1e_ref,
                   b1_ref, w2_ref, b2_ref, fw_ref, fb_ref, ea_out_ref, msg_ref):
    hs = hs_ref[...]
    hd = hd_ref[...]
    if first:
        d = ea_ref[...]
        kk = lax.broadcasted_iota(jnp.int32, (EBLK, GP), 1).astype(jnp.float32)
        diff = d - 0.1 * kk
        ea_in = jnp.exp(-_GCOEF * diff * diff)
    else:
        ea_in = ea_ref[...]
    pre = (_dot(hs, w1s_ref[...]) + _dot(hd, w1d_ref[...])
           + _dot(ea_in, w1e_ref[...]) + b1_ref[...])
    ea = _dot(_ssp(pre), w2_ref[...]) + b2_ref[...]
    hh = _dot(hs, fw_ref[...]) + fb_ref[...]
    rid = pl.program_id(0) * EBLK + lax.broadcasted_iota(jnp.int32, (EBLK, 1), 0)
    valid = (rid < E).astype(jnp.float32)
    ea_ref_out = ea * valid
    ea_out_ref[...] = ea_ref_out
    msg_ref[...] = hh * ea_ref_out


def _edge_mlp(first, hs, hd, ea_or_d, w1s, w1d, w1e, b1, w2, b2, fw, fb):
    ein = ea_or_d.shape[1]
    win = w1e.shape[0]
    return pl.pallas_call(
        functools.partial(_edge_mlp_body, first),
        grid=(EP // EBLK,),
        in_specs=[
            pl.BlockSpec((EBLK, C), lambda i: (i, 0)),
            pl.BlockSpec((EBLK, C), lambda i: (i, 0)),
            pl.BlockSpec((EBLK, ein), lambda i: (i, 0)),
            pl.BlockSpec((C, 2 * C), lambda i: (0, 0)),
            pl.BlockSpec((C, 2 * C), lambda i: (0, 0)),
            pl.BlockSpec((win, 2 * C), lambda i: (0, 0)),
            pl.BlockSpec((1, 2 * C), lambda i: (0, 0)),
            pl.BlockSpec((2 * C, C), lambda i: (0, 0)),
            pl.BlockSpec((1, C), lambda i: (0, 0)),
            pl.BlockSpec((C, C), lambda i: (0, 0)),
            pl.BlockSpec((1, C), lambda i: (0, 0)),
        ],
        out_specs=[
            pl.BlockSpec((EBLK, C), lambda i: (i, 0)),
            pl.BlockSpec((EBLK, C), lambda i: (i, 0)),
        ],
        out_shape=[
            jax.ShapeDtypeStruct((EP, C), jnp.float32),
            jax.ShapeDtypeStruct((EP, C), jnp.float32),
        ],
    )(hs, hd, ea_or_d, w1s, w1d, w1e, b1, w2, b2, fw, fb)


def _node_body(h_ref, m_ref, w1_ref, b1_ref, w2_ref, b2_ref, h_out_ref):
    m = m_ref[0] + m_ref[1]
    t = _ssp(_dot(m, w1_ref[...]) + b1_ref[...])
    h_out_ref[...] = h_ref[...] + _dot(t, w2_ref[...]) + b2_ref[...]


def _node(h, mout, w1, b1, w2, b2):
    return pl.pallas_call(
        _node_body,
        grid=(N // NBLK,),
        in_specs=[
            pl.BlockSpec((NBLK, C), lambda i: (i, 0)),
            pl.BlockSpec((NC, NBLK, C), lambda i: (0, i, 0)),
            pl.BlockSpec((C, C), lambda i: (0, 0)),
            pl.BlockSpec((1, C), lambda i: (0, 0)),
            pl.BlockSpec((C, C), lambda i: (0, 0)),
            pl.BlockSpec((1, C), lambda i: (0, 0)),
        ],
        out_specs=pl.BlockSpec((NBLK, C), lambda i: (i, 0)),
        out_shape=jax.ShapeDtypeStruct((N, C), jnp.float32),
    )(h, mout, w1, b1, w2, b2)


def _node_final_body(h_ref, m_ref, w1_ref, b1_ref, w2_ref, b2_ref,
                     hw1_ref, hb1_ref, hw2_ref, hb2_ref, out_ref):
    m = m_ref[0] + m_ref[1]
    t = _ssp(_dot(m, w1_ref[...]) + b1_ref[...])
    h = h_ref[...] + _dot(t, w2_ref[...]) + b2_ref[...]
    t2 = _ssp(_dot(h, hw1_ref[...]) + hb1_ref[...])
    hv = _dot(t2, hw2_ref[...]) + hb2_ref[...]
    part = jnp.sum(hv)

    @pl.when(pl.program_id(0) == 0)
    def _():
        out_ref[...] = jnp.zeros_like(out_ref)

    out_ref[...] += part[None, None]


def _node_final(h, mout, w1, b1, w2, b2, hw1, hb1, hw2, hb2):
    return pl.pallas_call(
        _node_final_body,
        grid=(N // NBLK,),
        in_specs=[
            pl.BlockSpec((NBLK, C), lambda i: (i, 0)),
            pl.BlockSpec((NC, NBLK, C), lambda i: (0, i, 0)),
            pl.BlockSpec((C, C), lambda i: (0, 0)),
            pl.BlockSpec((1, C), lambda i: (0, 0)),
            pl.BlockSpec((C, C), lambda i: (0, 0)),
            pl.BlockSpec((1, C), lambda i: (0, 0)),
            pl.BlockSpec((C, C // 2), lambda i: (0, 0)),
            pl.BlockSpec((1, C // 2), lambda i: (0, 0)),
            pl.BlockSpec((C // 2, 1), lambda i: (0, 0)),
            pl.BlockSpec((1, 1), lambda i: (0, 0)),
        ],
        out_specs=pl.BlockSpec((1, 1), lambda i: (0, 0)),
        out_shape=jax.ShapeDtypeStruct((1, 1), jnp.float32),
    )(h, mout, w1, b1, w2, b2, hw1, hb1, hw2, hb2)


@functools.cache
def _sc_mesh():
    return plsc.VectorSubcoreMesh(core_axis_name="c", subcore_axis_name="s",
                                  num_cores=NC, num_subcores=NS)


def _gather_body(h_hbm, src_hbm, dst_hbm, hs_out, hd_out,
                 hsp, idx, rows0, rows1, semg0, semg1, semw0, semw1):
    cid = lax.axis_index("c")
    sid = lax.axis_index("s")
    wid = sid * NC + cid

    cpi0 = pltpu.async_copy(src_hbm.at[pl.ds(wid * IRPW, IRPW)],
                            idx.at[pl.ds(0, IRPW)], semg0)
    cpi1 = pltpu.async_copy(dst_hbm.at[pl.ds(wid * IRPW, IRPW)],
                            idx.at[pl.ds(IRPW, IRPW)], semg1)
    cph = pltpu.async_copy(h_hbm.at[pl.ds(sid * NPS, NPS)],
                           hsp.at[pl.ds(sid * NPS, NPS)], semw0)
    cpi0.wait()
    cpi1.wait()
    cph.wait()
    plsc.subcore_barrier()

    rows = [rows0, rows1]
    semg = [semg0, semg1]
    semw = [semw0, semw1]
    outs = [hs_out, hd_out]
    steps = 2 * NCHUNK

    def issue_gathers(s, buf, sem):
        t, i = divmod(s, NCHUNK)
        cps = []
        for j in range(KJ):
            cps.append(pltpu.async_copy(
                hsp.at[idx.at[t * IRPW + i * KJ + j]],
                buf.at[pl.ds(j * 128, 128)], sem))
        return cps

    def issue_writeout(s, buf, sem):
        t, i = divmod(s, NCHUNK)
        return pltpu.async_copy(buf, outs[t].at[pl.ds(wid * EPW + i * CH, CH)],
                                sem)

    wc = [None, None]
    g = [issue_gathers(0, rows0, semg0), None]
    for s in range(steps):
        b = s % 2
        nb = 1 - b
        if s + 1 < steps:
            if wc[nb] is not None:
                wc[nb].wait()
            g[nb] = issue_gathers(s + 1, rows[nb], semg[nb])
        for cp in g[b]:
            cp.wait()
        wc[b] = issue_writeout(s, rows[b], semw[b])
    wc[0].wait()
    wc[1].wait()


def _gather(hpad, src2d, dst2d):
    return pl.kernel(
        _gather_body,
        out_type=[
            jax.ShapeDtypeStruct((EP, C), jnp.float32),
            jax.ShapeDtypeStruct((EP, C), jnp.float32),
        ],
        mesh=_sc_mesh(),
        compiler_params=pltpu.CompilerParams(use_tc_tiling_on_sc=False),
        scratch_types=[
            pltpu.VMEM_SHARED((NP, C), jnp.float32),
            pltpu.VMEM((2 * IRPW, 128), jnp.int32),
            pltpu.VMEM((CH, C), jnp.float32),
            pltpu.VMEM((CH, C), jnp.float32),
            pltpu.SemaphoreType.DMA,
            pltpu.SemaphoreType.DMA,
            pltpu.SemaphoreType.DMA,
            pltpu.SemaphoreType.DMA,
        ],
    )(hpad, src2d, dst2d)


def _scatter_body(msg_hbm, dst_hbm, zero_hbm, mout_hbm, macc, mbuf0, mbuf1,
                  idx, semm0, semm1, sema0, sema1):
    cid = lax.axis_index("c")
    sid = lax.axis_index("s")
    wid = sid * NC + cid

    cpi = pltpu.async_copy(dst_hbm.at[pl.ds(wid * IRPW, IRPW)], idx, semm0)
    cpm0 = pltpu.async_copy(msg_hbm.at[pl.ds(wid * EPW, CH)], mbuf0, semm1)
    pltpu.sync_copy(zero_hbm.at[pl.ds(sid * NPS, NPS)],
                    macc.at[pl.ds(sid * NPS, NPS)])
    cpi.wait()
    plsc.subcore_barrier()

    mbuf = [mbuf0, mbuf1]
    semm = [semm0, semm1]
    sema = [sema0, sema1]

    def issue_adds(i, buf, sem):
        cps = []
        for j in range(KJ):
            cps.append(pltpu.async_copy(
                buf.at[pl.ds(j * 128, 128)], macc.at[idx.at[i * KJ + j]],
                sem, add=True))
        return cps

    mc = [cpm0, None]
    ac = [None, None]
    for i in range(NCHUNK):
        b = i % 2
        nb = 1 - b
        if i + 1 < NCHUNK:
            if ac[nb] is not None:
                for cp in ac[nb]:
                    cp.wait()
                ac[nb] = None
            mc[nb] = pltpu.async_copy(
                msg_hbm.at[pl.ds(wid * EPW + (i + 1) * CH, CH)],
                mbuf[nb], semm[nb])
        mc[b].wait()
        ac[b] = issue_adds(i, mbuf[b], sema[b])
    for cps in ac:
        if cps is not None:
            for cp in cps:
                cp.wait()
    plsc.subcore_barrier()
    pltpu.sync_copy(macc.at[pl.ds(sid * NPS, NPS)],
                    mout_hbm.at[cid, pl.ds(sid * NPS, NPS)])


def _scatter(msg, dst2d, zeros_np):
    return pl.kernel(
        _scatter_body,
        out_type=jax.ShapeDtypeStruct((NC, NP, C), jnp.float32),
        mesh=_sc_mesh(),
        compiler_params=pltpu.CompilerParams(use_tc_tiling_on_sc=False),
        scratch_types=[
            pltpu.VMEM_SHARED((NP, C), jnp.float32),
            pltpu.VMEM((CH, C), jnp.float32),
            pltpu.VMEM((CH, C), jnp.float32),
            pltpu.VMEM((IRPW, 128), jnp.int32),
            pltpu.SemaphoreType.DMA,
            pltpu.SemaphoreType.DMA,
            pltpu.SemaphoreType.DMA,
            pltpu.SemaphoreType.DMA,
        ],
    )(msg, dst2d, zeros_np)


def kernel(z, edge_index, distances, params):
    src = edge_index[0].astype(jnp.int32)
    dst = edge_index[1].astype(jnp.int32)
    src2d = jnp.pad(src, (0, EP - E)).reshape(EP // 128, 128)
    dst2d = jnp.pad(dst, (0, EP - E)).reshape(EP // 128, 128)
    dpad = jnp.pad(distances.astype(jnp.float32), (0, EP - E)).reshape(EP, 1)
    zeros_np = jnp.zeros((NP, C), jnp.float32)

    h = _embed(z.reshape(N, 1).astype(jnp.int32), params["emb"])

    ea = dpad
    out = None
    for i in range(3):
        eu = params["eu"][i]
        it = params["it"][i]
        w1 = eu["W1"]
        w1s = w1[:C]
        w1d = w1[C:2 * C]
        w1e = w1[2 * C:]
        if i == 0:
            w1e = jnp.zeros((GP, 2 * C), jnp.float32).at[:G].set(w1e)
        b1 = eu["b1"].reshape(1, 2 * C)
        b2 = eu["b2"].reshape(1, C)
        fw = it["fW"]
        fb = it["fb"].reshape(1, C)

        hs, hd = _gather(jnp.pad(h, ((0, NP - N), (0, 0))), src2d, dst2d)
        ea, msg = _edge_mlp(i == 0, hs, hd, ea, w1s, w1d, w1e, b1,
                            eu["W2"], b2, fw, fb)
        mout = _scatter(msg, dst2d, zeros_np)

        m1w1 = it["m1W1"]
        m1b1 = it["m1b1"].reshape(1, C)
        m1w2 = it["m1W2"]
        m1b2 = it["m1b2"].reshape(1, C)
        if i < 2:
            h = _node(h, mout, m1w1, m1b1, m1w2, m1b2)
        else:
            hd_p = params["head"]
            out = _node_final(h, mout, m1w1, m1b1, m1w2, m1b2,
                              hd_p["W1"], hd_p["b1"].reshape(1, C // 2),
                              hd_p["W2"], hd_p["b2"].reshape(1, 1))
    return out


# speedup vs baseline: 5.7243x; 1.7886x over previous
import functools

import jax
import jax.numpy as jnp
from jax import lax
from jax.experimental import pallas as pl
from jax.experimental.pallas import tpu as pltpu
from jax.experimental.pallas import tpu_sc as plsc

N = 10000
E = 160000
C = 64
G = 150

NC = 2
NS = 16
NW = NC * NS
EP = 163840
EPW = EP // NW
CH = 512
KJ = CH // 128
NCHUNK = EPW // CH
IRPW = EPW // 128
NP = 10240
NPS = NP // NS

EBLK = 2048
EP2 = EP // 2
EBLK2 = EBLK // 2
NBLK = 2000
GP = 256

_LOG2 = 0.6931471805599453
_GCOEF = 0.1 ** 0.5


def _ssp(x):
    return jnp.maximum(x, 0.0) + jnp.log1p(jnp.exp(-jnp.abs(x))) - _LOG2


def _dot(a, b):
    return jnp.dot(a, b, preferred_element_type=jnp.float32)


def _embed_body(z_ref, emb_ref, h_ref):
    zb = z_ref[...]
    oh = (zb == lax.broadcasted_iota(jnp.int32, (NBLK, 100), 1)).astype(jnp.float32)
    h_ref[...] = _dot(oh, emb_ref[...])


def _embed(z2d, emb):
    return pl.pallas_call(
        _embed_body,
        grid=(N // NBLK,),
        in_specs=[
            pl.BlockSpec((NBLK, 1), lambda i: (i, 0)),
            pl.BlockSpec((100, C), lambda i: (0, 0)),
        ],
        out_specs=pl.BlockSpec((NBLK, C), lambda i: (i, 0)),
        out_shape=jax.ShapeDtypeStruct((N, C), jnp.float32),
    )(z2d, emb)


def _edge_mlp_body(first, hs_ref, hd_ref, dea_ref, dob_ref, w1s_ref, w1d_ref,
                   w1e_ref, b1_ref, w2_ref, b2_ref, fw_ref, fb_ref,
                   ea_out_ref, msg_ref):
    hs = hs_ref[...]
    hd = hd_ref[...]
    if first:
        de = dea_ref[...].reshape(EBLK2, 1)
        do = dob_ref[...].reshape(EBLK2, 1)
        kk = lax.broadcasted_iota(jnp.int32, (EBLK2, GP), 1).astype(jnp.float32)
        dfe = de - 0.1 * kk
        dfo = do - 0.1 * kk
        ge = jnp.exp(-_GCOEF * dfe * dfe)
        go = jnp.exp(-_GCOEF * dfo * dfo)
        pre_e = jnp.concatenate([_dot(ge, w1e_ref[...]),
                                 _dot(go, w1e_ref[...])], axis=1)
    else:
        pre_e = _dot(dea_ref[...], w1e_ref[...])
    pre = (_dot(hs, w1s_ref[...]) + _dot(hd, w1d_ref[...])
           + pre_e + b1_ref[...])
    ea = _dot(_ssp(pre), w2_ref[...]) + b2_ref[...]
    hh = _dot(hs, fw_ref[...]) + fb_ref[...]
    msg = hh * ea
    pid = pl.program_id(0)
    nfull = EP // EBLK - 2

    @pl.when(pid < nfull)
    def _():
        ea_out_ref[...] = ea
        msg_ref[...] = msg

    @pl.when(pid >= nfull)
    def _():
        rid = pid * EBLK2 + lax.broadcasted_iota(jnp.int32, (EBLK2, 1), 0)
        valid = (rid < E // 2).astype(jnp.float32)
        ea_out_ref[...] = ea * valid
        msg_ref[...] = msg * valid


def _edge_mlp(first, hs2, hd2, ea_or_de, dodd, w1s, w1d, w1e, b1, w2, b2,
              fw, fb):
    if first:
        ea_spec = pl.BlockSpec((EBLK2,), lambda i: (i,))
        do_spec = pl.BlockSpec((EBLK2,), lambda i: (i,))
    else:
        ea_spec = pl.BlockSpec((EBLK2, 2 * C), lambda i: (i, 0))
        do_spec = pl.BlockSpec((EBLK2,), lambda i: (0,))
    win = w1e.shape[0]
    wout = w1e.shape[1]
    return pl.pallas_call(
        functools.partial(_edge_mlp_body, first),
        grid=(EP // EBLK,),
        in_specs=[
            pl.BlockSpec((EBLK2, 2 * C), lambda i: (i, 0)),
            pl.BlockSpec((EBLK2, 2 * C), lambda i: (i, 0)),
            ea_spec,
            do_spec,
            pl.BlockSpec((2 * C, 4 * C), lambda i: (0, 0)),
            pl.BlockSpec((2 * C, 4 * C), lambda i: (0, 0)),
            pl.BlockSpec((win, wout), lambda i: (0, 0)),
            pl.BlockSpec((1, 4 * C), lambda i: (0, 0)),
            pl.BlockSpec((4 * C, 2 * C), lambda i: (0, 0)),
            pl.BlockSpec((1, 2 * C), lambda i: (0, 0)),
            pl.BlockSpec((2 * C, 2 * C), lambda i: (0, 0)),
            pl.BlockSpec((1, 2 * C), lambda i: (0, 0)),
        ],
        out_specs=[
            pl.BlockSpec((EBLK2, 2 * C), lambda i: (i, 0)),
            pl.BlockSpec((EBLK2, 2 * C), lambda i: (i, 0)),
        ],
        out_shape=[
            jax.ShapeDtypeStruct((EP2, 2 * C), jnp.float32),
            jax.ShapeDtypeStruct((EP2, 2 * C), jnp.float32),
        ],
    )(hs2, hd2, ea_or_de, dodd, w1s, w1d, w1e, b1, w2, b2, fw, fb)


def _node_body(h_ref, m_ref, w1_ref, b1_ref, w2_ref, b2_ref, h_out_ref):
    m = m_ref[0] + m_ref[1]
    t = _ssp(_dot(m, w1_ref[...]) + b1_ref[...])
    h_out_ref[...] = h_ref[...] + _dot(t, w2_ref[...]) + b2_ref[...]


def _node(h, mout, w1, b1, w2, b2):
    return pl.pallas_call(
        _node_body,
        grid=(N // NBLK,),
        in_specs=[
            pl.BlockSpec((NBLK, C), lambda i: (i, 0)),
            pl.BlockSpec((NC, NBLK, C), lambda i: (0, i, 0)),
            pl.BlockSpec((C, C), lambda i: (0, 0)),
            pl.BlockSpec((1, C), lambda i: (0, 0)),
            pl.BlockSpec((C, C), lambda i: (0, 0)),
            pl.BlockSpec((1, C), lambda i: (0, 0)),
        ],
        out_specs=pl.BlockSpec((NBLK, C), lambda i: (i, 0)),
        out_shape=jax.ShapeDtypeStruct((N, C), jnp.float32),
    )(h, mout, w1, b1, w2, b2)


def _node_final_body(h_ref, m_ref, w1_ref, b1_ref, w2_ref, b2_ref,
                     hw1_ref, hb1_ref, hw2_ref, hb2_ref, out_ref):
    m = m_ref[0] + m_ref[1]
    t = _ssp(_dot(m, w1_ref[...]) + b1_ref[...])
    h = h_ref[...] + _dot(t, w2_ref[...]) + b2_ref[...]
    t2 = _ssp(_dot(h, hw1_ref[...]) + hb1_ref[...])
    hv = _dot(t2, hw2_ref[...]) + hb2_ref[...]
    part = jnp.sum(hv)

    @pl.when(pl.program_id(0) == 0)
    def _():
        out_ref[...] = jnp.zeros_like(out_ref)

    out_ref[...] += part[None, None]


def _node_final(h, mout, w1, b1, w2, b2, hw1, hb1, hw2, hb2):
    return pl.pallas_call(
        _node_final_body,
        grid=(N // NBLK,),
        in_specs=[
            pl.BlockSpec((NBLK, C), lambda i: (i, 0)),
            pl.BlockSpec((NC, NBLK, C), lambda i: (0, i, 0)),
            pl.BlockSpec((C, C), lambda i: (0, 0)),
            pl.BlockSpec((1, C), lambda i: (0, 0)),
            pl.BlockSpec((C, C), lambda i: (0, 0)),
            pl.BlockSpec((1, C), lambda i: (0, 0)),
            pl.BlockSpec((C, C // 2), lambda i: (0, 0)),
            pl.BlockSpec((1, C // 2), lambda i: (0, 0)),
            pl.BlockSpec((C // 2, 1), lambda i: (0, 0)),
            pl.BlockSpec((1, 1), lambda i: (0, 0)),
        ],
        out_specs=pl.BlockSpec((1, 1), lambda i: (0, 0)),
        out_shape=jax.ShapeDtypeStruct((1, 1), jnp.float32),
    )(h, mout, w1, b1, w2, b2, hw1, hb1, hw2, hb2)


@functools.cache
def _sc_mesh():
    return plsc.VectorSubcoreMesh(core_axis_name="c", subcore_axis_name="s",
                                  num_cores=NC, num_subcores=NS)


def _gather_body(h_hbm, src_hbm, dst_hbm, hs_out, hd_out,
                 hsp, idx, rows0, rows1, semg0, semg1, semw0, semw1):
    cid = lax.axis_index("c")
    sid = lax.axis_index("s")
    wid = sid * NC + cid

    cpi0 = pltpu.async_copy(src_hbm.at[pl.ds(wid * IRPW, IRPW)],
                            idx.at[pl.ds(0, IRPW)], semg0)
    cpi1 = pltpu.async_copy(dst_hbm.at[pl.ds(wid * IRPW, IRPW)],
                            idx.at[pl.ds(IRPW, IRPW)], semg1)
    cph = pltpu.async_copy(h_hbm.at[pl.ds(sid * NPS, NPS)],
                           hsp.at[pl.ds(sid * NPS, NPS)], semw0)
    cpi0.wait()
    cpi1.wait()
    cph.wait()
    plsc.subcore_barrier()

    rows = [rows0, rows1]
    semg = [semg0, semg1]
    semw = [semw0, semw1]
    outs = [hs_out, hd_out]
    steps = 2 * NCHUNK

    def issue_gathers(s, buf, sem):
        t, i = divmod(s, NCHUNK)
        cps = []
        for j in range(KJ):
            cps.append(pltpu.async_copy(
                hsp.at[idx.at[t * IRPW + i * KJ + j]],
                buf.at[pl.ds(j * 128, 128)], sem))
        return cps

    def issue_writeout(s, buf, sem):
        t, i = divmod(s, NCHUNK)
        return pltpu.async_copy(buf, outs[t].at[pl.ds(wid * EPW + i * CH, CH)],
                                sem)

    wc = [None, None]
    g = [issue_gathers(0, rows0, semg0), None]
    for s in range(steps):
        b = s % 2
        nb = 1 - b
        if s + 1 < steps:
            if wc[nb] is not None:
                wc[nb].wait()
            g[nb] = issue_gathers(s + 1, rows[nb], semg[nb])
        for cp in g[b]:
            cp.wait()
        wc[b] = issue_writeout(s, rows[b], semw[b])
    wc[0].wait()
    wc[1].wait()


def _gather(hpad, src2d, dst2d):
    return pl.kernel(
        _gather_body,
        out_type=[
            jax.ShapeDtypeStruct((EP, C), jnp.float32),
            jax.ShapeDtypeStruct((EP, C), jnp.float32),
        ],
        mesh=_sc_mesh(),
        compiler_params=pltpu.CompilerParams(use_tc_tiling_on_sc=False),
        scratch_types=[
            pltpu.VMEM_SHARED((NP, C), jnp.float32),
            pltpu.VMEM((2 * IRPW, 128), jnp.int32),
            pltpu.VMEM((CH, C), jnp.float32),
            pltpu.VMEM((CH, C), jnp.float32),
            pltpu.SemaphoreType.DMA,
            pltpu.SemaphoreType.DMA,
            pltpu.SemaphoreType.DMA,
            pltpu.SemaphoreType.DMA,
        ],
    )(hpad, src2d, dst2d)


def _scatter_body(msg_hbm, dst_hbm, zero_hbm, mout_hbm, macc, mbuf0, mbuf1,
                  idx, semm0, semm1, sema0, sema1):
    cid = lax.axis_index("c")
    sid = lax.axis_index("s")
    wid = sid * NC + cid

    cpi = pltpu.async_copy(dst_hbm.at[pl.ds(wid * IRPW, IRPW)], idx, semm0)
    cpm0 = pltpu.async_copy(msg_hbm.at[pl.ds(wid * EPW, CH)], mbuf0, semm1)
    pltpu.sync_copy(zero_hbm.at[pl.ds(sid * NPS, NPS)],
                    macc.at[pl.ds(sid * NPS, NPS)])
    cpi.wait()
    plsc.subcore_barrier()

    mbuf = [mbuf0, mbuf1]
    semm = [semm0, semm1]
    sema = [sema0, sema1]

    def issue_adds(i, buf, sem):
        cps = []
        for j in range(KJ):
            cps.append(pltpu.async_copy(
                buf.at[pl.ds(j * 128, 128)],
                macc.at[idx.at[i * KJ + j]], sem, add=True))
        return cps

    mc = [cpm0, None]
    ac = [None, None]
    for i in range(NCHUNK):
        b = i % 2
        nb = 1 - b
        if i + 1 < NCHUNK:
            if ac[nb] is not None:
                for cp in ac[nb]:
                    cp.wait()
                ac[nb] = None
            mc[nb] = pltpu.async_copy(
                msg_hbm.at[pl.ds(wid * EPW + (i + 1) * CH, CH)],
                mbuf[nb], semm[nb])
        mc[b].wait()
        ac[b] = issue_adds(i, mbuf[b], sema[b])
    for cps in ac:
        if cps is not None:
            for cp in cps:
                cp.wait()
    plsc.subcore_barrier()
    pltpu.sync_copy(macc.at[pl.ds(sid * NPS, NPS)],
                    mout_hbm.at[cid, pl.ds(sid * NPS, NPS)])


def _scatter(msg, dst2d, zeros_np):
    return pl.kernel(
        _scatter_body,
        out_type=jax.ShapeDtypeStruct((NC, NP, C), jnp.float32),
        mesh=_sc_mesh(),
        compiler_params=pltpu.CompilerParams(use_tc_tiling_on_sc=False),
        scratch_types=[
            pltpu.VMEM_SHARED((NP, C), jnp.float32),
            pltpu.VMEM((CH, C), jnp.float32),
            pltpu.VMEM((CH, C), jnp.float32),
            pltpu.VMEM((IRPW, 128), jnp.int32),
            pltpu.SemaphoreType.DMA,
            pltpu.SemaphoreType.DMA,
            pltpu.SemaphoreType.DMA,
            pltpu.SemaphoreType.DMA,
        ],
    )(msg, dst2d, zeros_np)


def _bd(w):
    fin, fout = w.shape
    return (jnp.zeros((2 * fin, 2 * fout), jnp.float32)
            .at[:fin, :fout].set(w).at[fin:, fout:].set(w))


def kernel(z, edge_index, distances, params):
    src = edge_index[0].astype(jnp.int32)
    dst = edge_index[1].astype(jnp.int32)
    src2d = jnp.pad(src, (0, EP - E)).reshape(EP // 128, 128)
    dst2d = jnp.pad(dst, (0, EP - E)).reshape(EP // 128, 128)
    dpad = jnp.pad(distances.astype(jnp.float32), (0, EP - E)).reshape(EP2, 2)
    d_even = dpad[:, 0]
    d_odd = dpad[:, 1]
    zeros_np = jnp.zeros((NP, C), jnp.float32)

    h = _embed(z.reshape(N, 1).astype(jnp.int32), params["emb"])

    ea = d_even
    out = None
    for i in range(3):
        eu = params["eu"][i]
        it = params["it"][i]
        w1 = eu["W1"]
        w1s = _bd(w1[:C])
        w1d = _bd(w1[C:2 * C])
        if i == 0:
            w1e = jnp.zeros((GP, 2 * C), jnp.float32).at[:G].set(w1[2 * C:])
        else:
            w1e = _bd(w1[2 * C:])
        b1 = jnp.tile(eu["b1"], 2).reshape(1, 4 * C)
        w2 = _bd(eu["W2"])
        b2 = jnp.tile(eu["b2"], 2).reshape(1, 2 * C)
        fw = _bd(it["fW"])
        fb = jnp.tile(it["fb"], 2).reshape(1, 2 * C)

        hs, hd = _gather(jnp.pad(h, ((0, NP - N), (0, 0))), src2d, dst2d)
        ea, msg2 = _edge_mlp(i == 0, hs.reshape(EP2, 2 * C),
                             hd.reshape(EP2, 2 * C), ea, d_odd,
                             w1s, w1d, w1e, b1, w2, b2, fw, fb)
        mout = _scatter(msg2.reshape(EP, C), dst2d, zeros_np)

        m1w1 = it["m1W1"]
        m1b1 = it["m1b1"].reshape(1, C)
        m1w2 = it["m1W2"]
        m1b2 = it["m1b2"].reshape(1, C)
        if i < 2:
            h = _node(h, mout, m1w1, m1b1, m1w2, m1b2)
        else:
            hd_p = params["head"]
            out = _node_final(h, mout, m1w1, m1b1, m1w2, m1b2,
                              hd_p["W1"], hd_p["b1"].reshape(1, C // 2),
                              hd_p["W2"], hd_p["b2"].reshape(1, 1))
    return out


# speedup vs baseline: 6.1068x; 1.0668x over previous
import functools

import jax
import jax.numpy as jnp
from jax import lax
from jax.experimental import pallas as pl
from jax.experimental.pallas import tpu as pltpu
from jax.experimental.pallas import tpu_sc as plsc

N = 10000
E = 160000
C = 64
G = 150

NC = 2
NS = 16
NW = NC * NS
EP = 163840
EPW = EP // NW
CH = 512
KJ = CH // 128
NCHUNK = EPW // CH
IRPW = EPW // 128
NP = 10240
NPS = NP // NS

EBLK = 2048
EP2 = EP // 2
EBLK2 = EBLK // 2
NBLK = 2000
GP = 256

_LOG2 = 0.6931471805599453
_GCOEF = 0.1 ** 0.5


def _ssp(x):
    return jnp.maximum(x, 0.0) + jnp.log1p(jnp.exp(-jnp.abs(x))) - _LOG2


def _dot(a, b):
    return jnp.dot(a, b, preferred_element_type=jnp.float32)


def _embed_body(z_ref, emb_ref, h_ref):
    zb = z_ref[...]
    oh = (zb == lax.broadcasted_iota(jnp.int32, (NBLK, 100), 1)).astype(jnp.float32)
    h_ref[...] = _dot(oh, emb_ref[...])


def _embed(z2d, emb):
    return pl.pallas_call(
        _embed_body,
        grid=(N // NBLK,),
        in_specs=[
            pl.BlockSpec((NBLK, 1), lambda i: (i, 0)),
            pl.BlockSpec((100, C), lambda i: (0, 0)),
        ],
        out_specs=pl.BlockSpec((NBLK, C), lambda i: (i, 0)),
        out_shape=jax.ShapeDtypeStruct((N, C), jnp.float32),
    )(z2d, emb)


def _edge_mlp_body(first, hs_ref, hd_ref, dea_ref, dob_ref, w1s_ref, w1d_ref,
                   w1e_ref, b1_ref, w2_ref, b2_ref, fw_ref, fb_ref,
                   ea_out_ref, msg_ref):
    hs = hs_ref[...]
    hd = hd_ref[...]
    if first:
        de = dea_ref[...].reshape(EBLK2, 1)
        do = dob_ref[...].reshape(EBLK2, 1)
        kk = lax.broadcasted_iota(jnp.int32, (EBLK2, GP), 1).astype(jnp.float32)
        dfe = de - 0.1 * kk
        dfo = do - 0.1 * kk
        ge = jnp.exp(-_GCOEF * dfe * dfe)
        go = jnp.exp(-_GCOEF * dfo * dfo)
        pre_e = jnp.concatenate([_dot(ge, w1e_ref[...]),
                                 _dot(go, w1e_ref[...])], axis=1)
    else:
        pre_e = _dot(dea_ref[...], w1e_ref[...])
    pre = (_dot(hs, w1s_ref[...]) + _dot(hd, w1d_ref[...])
           + pre_e + b1_ref[...])
    ea = _dot(_ssp(pre), w2_ref[...]) + b2_ref[...]
    hh = _dot(hs, fw_ref[...]) + fb_ref[...]
    msg = hh * ea
    pid = pl.program_id(0)
    nfull = EP // EBLK - 2

    @pl.when(pid < nfull)
    def _():
        ea_out_ref[...] = ea
        msg_ref[...] = msg

    @pl.when(pid >= nfull)
    def _():
        rid = pid * EBLK2 + lax.broadcasted_iota(jnp.int32, (EBLK2, 1), 0)
        valid = (rid < E // 2).astype(jnp.float32)
        ea_out_ref[...] = ea * valid
        msg_ref[...] = msg * valid


def _edge_mlp(first, hs2, hd2, ea_or_de, dodd, w1s, w1d, w1e, b1, w2, b2,
              fw, fb):
    if first:
        ea_spec = pl.BlockSpec((EBLK2,), lambda i: (i,))
        do_spec = pl.BlockSpec((EBLK2,), lambda i: (i,))
    else:
        ea_spec = pl.BlockSpec((EBLK2, 2 * C), lambda i: (i, 0))
        do_spec = pl.BlockSpec((EBLK2,), lambda i: (0,))
    win = w1e.shape[0]
    wout = w1e.shape[1]
    return pl.pallas_call(
        functools.partial(_edge_mlp_body, first),
        grid=(EP // EBLK,),
        in_specs=[
            pl.BlockSpec((EBLK2, 2 * C), lambda i: (i, 0)),
            pl.BlockSpec((EBLK2, 2 * C), lambda i: (i, 0)),
            ea_spec,
            do_spec,
            pl.BlockSpec((2 * C, 4 * C), lambda i: (0, 0)),
            pl.BlockSpec((2 * C, 4 * C), lambda i: (0, 0)),
            pl.BlockSpec((win, wout), lambda i: (0, 0)),
            pl.BlockSpec((1, 4 * C), lambda i: (0, 0)),
            pl.BlockSpec((4 * C, 2 * C), lambda i: (0, 0)),
            pl.BlockSpec((1, 2 * C), lambda i: (0, 0)),
            pl.BlockSpec((2 * C, 2 * C), lambda i: (0, 0)),
            pl.BlockSpec((1, 2 * C), lambda i: (0, 0)),
        ],
        out_specs=[
            pl.BlockSpec((EBLK2, 2 * C), lambda i: (i, 0)),
            pl.BlockSpec((EBLK2, 2 * C), lambda i: (i, 0)),
        ],
        out_shape=[
            jax.ShapeDtypeStruct((EP2, 2 * C), jnp.float32),
            jax.ShapeDtypeStruct((EP2, 2 * C), jnp.float32),
        ],
    )(hs2, hd2, ea_or_de, dodd, w1s, w1d, w1e, b1, w2, b2, fw, fb)


def _node_body(h_ref, m_ref, w1_ref, b1_ref, w2_ref, b2_ref, h_out_ref):
    m = m_ref[0] + m_ref[1]
    t = _ssp(_dot(m, w1_ref[...]) + b1_ref[...])
    h_out_ref[...] = h_ref[...] + _dot(t, w2_ref[...]) + b2_ref[...]


def _node(h, mout, w1, b1, w2, b2):
    return pl.pallas_call(
        _node_body,
        grid=(N // NBLK,),
        in_specs=[
            pl.BlockSpec((NBLK, C), lambda i: (i, 0)),
            pl.BlockSpec((NC, NBLK, C), lambda i: (0, i, 0)),
            pl.BlockSpec((C, C), lambda i: (0, 0)),
            pl.BlockSpec((1, C), lambda i: (0, 0)),
            pl.BlockSpec((C, C), lambda i: (0, 0)),
            pl.BlockSpec((1, C), lambda i: (0, 0)),
        ],
        out_specs=pl.BlockSpec((NBLK, C), lambda i: (i, 0)),
        out_shape=jax.ShapeDtypeStruct((N, C), jnp.float32),
    )(h, mout, w1, b1, w2, b2)


def _node_final_body(h_ref, m_ref, w1_ref, b1_ref, w2_ref, b2_ref,
                     hw1_ref, hb1_ref, hw2_ref, hb2_ref, out_ref):
    m = m_ref[0] + m_ref[1]
    t = _ssp(_dot(m, w1_ref[...]) + b1_ref[...])
    h = h_ref[...] + _dot(t, w2_ref[...]) + b2_ref[...]
    t2 = _ssp(_dot(h, hw1_ref[...]) + hb1_ref[...])
    hv = _dot(t2, hw2_ref[...]) + hb2_ref[...]
    part = jnp.sum(hv)

    @pl.when(pl.program_id(0) == 0)
    def _():
        out_ref[...] = jnp.zeros_like(out_ref)

    out_ref[...] += part[None, None]


def _node_final(h, mout, w1, b1, w2, b2, hw1, hb1, hw2, hb2):
    return pl.pallas_call(
        _node_final_body,
        grid=(N // NBLK,),
        in_specs=[
            pl.BlockSpec((NBLK, C), lambda i: (i, 0)),
            pl.BlockSpec((NC, NBLK, C), lambda i: (0, i, 0)),
            pl.BlockSpec((C, C), lambda i: (0, 0)),
            pl.BlockSpec((1, C), lambda i: (0, 0)),
            pl.BlockSpec((C, C), lambda i: (0, 0)),
            pl.BlockSpec((1, C), lambda i: (0, 0)),
            pl.BlockSpec((C, C // 2), lambda i: (0, 0)),
            pl.BlockSpec((1, C // 2), lambda i: (0, 0)),
            pl.BlockSpec((C // 2, 1), lambda i: (0, 0)),
            pl.BlockSpec((1, 1), lambda i: (0, 0)),
        ],
        out_specs=pl.BlockSpec((1, 1), lambda i: (0, 0)),
        out_shape=jax.ShapeDtypeStruct((1, 1), jnp.float32),
    )(h, mout, w1, b1, w2, b2, hw1, hb1, hw2, hb2)


@functools.cache
def _sc_mesh():
    return plsc.VectorSubcoreMesh(core_axis_name="c", subcore_axis_name="s",
                                  num_cores=NC, num_subcores=NS)


def _gather_body(h_hbm, eidx_hbm, hs_out, hd_out,
                 hsp, idx, rows0, rows1, semg0, semg1, semw0, semw1):
    cid = lax.axis_index("c")
    sid = lax.axis_index("s")
    wid = sid * NC + cid

    cpi0 = pltpu.async_copy(eidx_hbm.at[0, pl.ds(wid * IRPW, IRPW)],
                            idx.at[pl.ds(0, IRPW)], semg0)
    cpi1 = pltpu.async_copy(eidx_hbm.at[1, pl.ds(wid * IRPW, IRPW)],
                            idx.at[pl.ds(IRPW, IRPW)], semg1)
    cph = pltpu.async_copy(h_hbm.at[pl.ds(sid * NPS, NPS)],
                           hsp.at[pl.ds(sid * NPS, NPS)], semw0)
    cpi0.wait()
    cpi1.wait()
    cph.wait()
    plsc.subcore_barrier()

    rows = [rows0, rows1]
    semg = [semg0, semg1]
    semw = [semw0, semw1]
    outs = [hs_out, hd_out]
    steps = 2 * NCHUNK

    def issue_gathers(s, buf, sem):
        t, i = divmod(s, NCHUNK)
        cps = []
        for j in range(KJ):
            cps.append(pltpu.async_copy(
                hsp.at[idx.at[t * IRPW + i * KJ + j]],
                buf.at[pl.ds(j * 128, 128)], sem))
        return cps

    def issue_writeout(s, buf, sem):
        t, i = divmod(s, NCHUNK)
        return pltpu.async_copy(buf, outs[t].at[pl.ds(wid * EPW + i * CH, CH)],
                                sem)

    wc = [None, None]
    g = [issue_gathers(0, rows0, semg0), None]
    for s in range(steps):
        b = s % 2
        nb = 1 - b
        if s + 1 < steps:
            if wc[nb] is not None:
                wc[nb].wait()
            g[nb] = issue_gathers(s + 1, rows[nb], semg[nb])
        for cp in g[b]:
            cp.wait()
        wc[b] = issue_writeout(s, rows[b], semw[b])
    wc[0].wait()
    wc[1].wait()


def _gather(hpad, eidx3d):
    return pl.kernel(
        _gather_body,
        out_type=[
            jax.ShapeDtypeStruct((EP, C), jnp.float32),
            jax.ShapeDtypeStruct((EP, C), jnp.float32),
        ],
        mesh=_sc_mesh(),
        compiler_params=pltpu.CompilerParams(use_tc_tiling_on_sc=False),
        scratch_types=[
            pltpu.VMEM_SHARED((NP, C), jnp.float32),
            pltpu.VMEM((2 * IRPW, 128), jnp.int32),
            pltpu.VMEM((CH, C), jnp.float32),
            pltpu.VMEM((CH, C), jnp.float32),
            pltpu.SemaphoreType.DMA,
            pltpu.SemaphoreType.DMA,
            pltpu.SemaphoreType.DMA,
            pltpu.SemaphoreType.DMA,
        ],
    )(hpad, eidx3d)


def _scatter_body(msg_hbm, eidx_hbm, zero_hbm, mout_hbm, macc, mbuf0, mbuf1,
                  idx, semm0, semm1, sema0, sema1):
    cid = lax.axis_index("c")
    sid = lax.axis_index("s")
    wid = sid * NC + cid

    cpi = pltpu.async_copy(eidx_hbm.at[1, pl.ds(wid * IRPW, IRPW)], idx, semm0)
    cpm0 = pltpu.async_copy(msg_hbm.at[pl.ds(wid * EPW, CH)], mbuf0, semm1)
    pltpu.sync_copy(zero_hbm.at[pl.ds(sid * NPS, NPS)],
                    macc.at[pl.ds(sid * NPS, NPS)])
    cpi.wait()
    plsc.subcore_barrier()

    mbuf = [mbuf0, mbuf1]
    semm = [semm0, semm1]
    sema = [sema0, sema1]

    def issue_adds(i, buf, sem):
        cps = []
        for j in range(KJ):
            cps.append(pltpu.async_copy(
                buf.at[pl.ds(j * 128, 128)],
                macc.at[idx.at[i * KJ + j]], sem, add=True))
        return cps

    mc = [cpm0, None]
    ac = [None, None]
    for i in range(NCHUNK):
        b = i % 2
        nb = 1 - b
        if i + 1 < NCHUNK:
            if ac[nb] is not None:
                for cp in ac[nb]:
                    cp.wait()
                ac[nb] = None
            mc[nb] = pltpu.async_copy(
                msg_hbm.at[pl.ds(wid * EPW + (i + 1) * CH, CH)],
                mbuf[nb], semm[nb])
        mc[b].wait()
        ac[b] = issue_adds(i, mbuf[b], sema[b])
    for cps in ac:
        if cps is not None:
            for cp in cps:
                cp.wait()
    plsc.subcore_barrier()
    pltpu.sync_copy(macc.at[pl.ds(sid * NPS, NPS)],
                    mout_hbm.at[cid, pl.ds(sid * NPS, NPS)])


def _scatter(msg, eidx3d, zeros_np):
    return pl.kernel(
        _scatter_body,
        out_type=jax.ShapeDtypeStruct((NC, NP, C), jnp.float32),
        mesh=_sc_mesh(),
        compiler_params=pltpu.CompilerParams(use_tc_tiling_on_sc=False),
        scratch_types=[
            pltpu.VMEM_SHARED((NP, C), jnp.float32),
            pltpu.VMEM((CH, C), jnp.float32),
            pltpu.VMEM((CH, C), jnp.float32),
            pltpu.VMEM((IRPW, 128), jnp.int32),
            pltpu.SemaphoreType.DMA,
            pltpu.SemaphoreType.DMA,
            pltpu.SemaphoreType.DMA,
            pltpu.SemaphoreType.DMA,
        ],
    )(msg, eidx3d, zeros_np)


def _bd(w):
    fin, fout = w.shape
    return (jnp.zeros((2 * fin, 2 * fout), jnp.float32)
            .at[:fin, :fout].set(w).at[fin:, fout:].set(w))


def kernel(z, edge_index, distances, params):
    eidx3d = jnp.pad(edge_index.astype(jnp.int32),
                     ((0, 0), (0, EP - E))).reshape(2, EP // 128, 128)
    dpad = jnp.pad(distances.astype(jnp.float32), (0, EP - E))
    d_even = lax.slice(dpad, (0,), (EP - 1,), (2,))
    d_odd = lax.slice(dpad, (1,), (EP,), (2,))
    zeros_np = jnp.zeros((NP, C), jnp.float32)

    h = _embed(z.reshape(N, 1).astype(jnp.int32), params["emb"])

    ea = d_even
    out = None
    for i in range(3):
        eu = params["eu"][i]
        it = params["it"][i]
        w1 = eu["W1"]
        w1s = _bd(w1[:C])
        w1d = _bd(w1[C:2 * C])
        if i == 0:
            w1e = jnp.zeros((GP, 2 * C), jnp.float32).at[:G].set(w1[2 * C:])
        else:
            w1e = _bd(w1[2 * C:])
        b1 = jnp.tile(eu["b1"], 2).reshape(1, 4 * C)
        w2 = _bd(eu["W2"])
        b2 = jnp.tile(eu["b2"], 2).reshape(1, 2 * C)
        fw = _bd(it["fW"])
        fb = jnp.tile(it["fb"], 2).reshape(1, 2 * C)

        hs, hd = _gather(jnp.pad(h, ((0, NP - N), (0, 0))), eidx3d)
        ea, msg2 = _edge_mlp(i == 0, hs.reshape(EP2, 2 * C),
                             hd.reshape(EP2, 2 * C), ea, d_odd,
                             w1s, w1d, w1e, b1, w2, b2, fw, fb)
        mout = _scatter(msg2.reshape(EP, C), eidx3d, zeros_np)

        m1w1 = it["m1W1"]
        m1b1 = it["m1b1"].reshape(1, C)
        m1w2 = it["m1W2"]
        m1b2 = it["m1b2"].reshape(1, C)
        if i < 2:
            h = _node(h, mout, m1w1, m1b1, m1w2, m1b2)
        else:
            hd_p = params["head"]
            out = _node_final(h, mout, m1w1, m1b1, m1w2, m1b2,
                              hd_p["W1"], hd_p["b1"].reshape(1, C // 2),
                              hd_p["W2"], hd_p["b2"].reshape(1, 1))
    return out


# speedup vs baseline: 6.6613x; 1.0908x over previous
import functools

import jax
import jax.numpy as jnp
from jax import lax
from jax.experimental import pallas as pl
from jax.experimental.pallas import tpu as pltpu
from jax.experimental.pallas import tpu_sc as plsc

N = 10000
E = 160000
C = 64
G = 150

NC = 2
NS = 16
NW = NC * NS
EP = 163840
EPW = EP // NW
CH = 512
KJ = CH // 128
NCHUNK = EPW // CH
IRPW = EPW // 128
NP = 10240
NPS = NP // NS

EBLK = 4096
EP2 = EP // 2
EBLK2 = EBLK // 2
NBLK = 2000
GP = 256

_LOG2 = 0.6931471805599453
_GCOEF = 0.1 ** 0.5


def _ssp(x):
    return jnp.maximum(x, 0.0) + jnp.log1p(jnp.exp(-jnp.abs(x))) - _LOG2


def _dot(a, b):
    return jnp.dot(a, b, preferred_element_type=jnp.float32)


def _embed_body(z_ref, emb_ref, h_ref):
    zb = z_ref[...]
    oh = (zb == lax.broadcasted_iota(jnp.int32, (NBLK, 100), 1)).astype(jnp.float32)
    h_ref[...] = _dot(oh, emb_ref[...])


def _embed(z2d, emb):
    return pl.pallas_call(
        _embed_body,
        grid=(N // NBLK,),
        in_specs=[
            pl.BlockSpec((NBLK, 1), lambda i: (i, 0)),
            pl.BlockSpec((100, C), lambda i: (0, 0)),
        ],
        out_specs=pl.BlockSpec((NBLK, C), lambda i: (i, 0)),
        out_shape=jax.ShapeDtypeStruct((N, C), jnp.float32),
    )(z2d, emb)


def _edge_mlp_body(first, hs_ref, hd_ref, dea_ref, dob_ref, w1s_ref, w1d_ref,
                   w1e_ref, b1_ref, w2_ref, b2_ref, fw_ref, fb_ref,
                   ea_out_ref, msg_ref):
    hs = hs_ref[...]
    hd = hd_ref[...]
    if first:
        de = dea_ref[...].reshape(EBLK2, 1)
        do = dob_ref[...].reshape(EBLK2, 1)
        kk = lax.broadcasted_iota(jnp.int32, (EBLK2, GP), 1).astype(jnp.float32)
        dfe = de - 0.1 * kk
        dfo = do - 0.1 * kk
        ge = jnp.exp(-_GCOEF * dfe * dfe)
        go = jnp.exp(-_GCOEF * dfo * dfo)
        pre_e = jnp.concatenate([_dot(ge, w1e_ref[...]),
                                 _dot(go, w1e_ref[...])], axis=1)
    else:
        pre_e = _dot(dea_ref[...], w1e_ref[...])
    pre = (_dot(hs, w1s_ref[...]) + _dot(hd, w1d_ref[...])
           + pre_e + b1_ref[...])
    ea = _dot(_ssp(pre), w2_ref[...]) + b2_ref[...]
    hh = _dot(hs, fw_ref[...]) + fb_ref[...]
    msg = hh * ea
    pid = pl.program_id(0)
    nfull = EP // EBLK - 2

    @pl.when(pid < nfull)
    def _():
        ea_out_ref[...] = ea
        msg_ref[...] = msg

    @pl.when(pid >= nfull)
    def _():
        rid = pid * EBLK2 + lax.broadcasted_iota(jnp.int32, (EBLK2, 1), 0)
        valid = (rid < E // 2).astype(jnp.float32)
        ea_out_ref[...] = ea * valid
        msg_ref[...] = msg * valid


def _edge_mlp(first, hs2, hd2, ea_or_de, dodd, w1s, w1d, w1e, b1, w2, b2,
              fw, fb):
    if first:
        ea_spec = pl.BlockSpec((EBLK2,), lambda i: (i,))
        do_spec = pl.BlockSpec((EBLK2,), lambda i: (i,))
    else:
        ea_spec = pl.BlockSpec((EBLK2, 2 * C), lambda i: (i, 0))
        do_spec = pl.BlockSpec((EBLK2,), lambda i: (0,))
    win = w1e.shape[0]
    wout = w1e.shape[1]
    return pl.pallas_call(
        functools.partial(_edge_mlp_body, first),
        grid=(EP // EBLK,),
        in_specs=[
            pl.BlockSpec((EBLK2, 2 * C), lambda i: (i, 0)),
            pl.BlockSpec((EBLK2, 2 * C), lambda i: (i, 0)),
            ea_spec,
            do_spec,
            pl.BlockSpec((2 * C, 4 * C), lambda i: (0, 0)),
            pl.BlockSpec((2 * C, 4 * C), lambda i: (0, 0)),
            pl.BlockSpec((win, wout), lambda i: (0, 0)),
            pl.BlockSpec((1, 4 * C), lambda i: (0, 0)),
            pl.BlockSpec((4 * C, 2 * C), lambda i: (0, 0)),
            pl.BlockSpec((1, 2 * C), lambda i: (0, 0)),
            pl.BlockSpec((2 * C, 2 * C), lambda i: (0, 0)),
            pl.BlockSpec((1, 2 * C), lambda i: (0, 0)),
        ],
        out_specs=[
            pl.BlockSpec((EBLK2, 2 * C), lambda i: (i, 0)),
            pl.BlockSpec((EBLK2, 2 * C), lambda i: (i, 0)),
        ],
        out_shape=[
            jax.ShapeDtypeStruct((EP2, 2 * C), jnp.float32),
            jax.ShapeDtypeStruct((EP2, 2 * C), jnp.float32),
        ],
    )(hs2, hd2, ea_or_de, dodd, w1s, w1d, w1e, b1, w2, b2, fw, fb)


def _node_body(h_ref, m_ref, w1_ref, b1_ref, w2_ref, b2_ref, h_out_ref):
    m = m_ref[0] + m_ref[1]
    t = _ssp(_dot(m, w1_ref[...]) + b1_ref[...])
    h_out_ref[...] = h_ref[...] + _dot(t, w2_ref[...]) + b2_ref[...]


def _node(h, mout, w1, b1, w2, b2):
    return pl.pallas_call(
        _node_body,
        grid=(N // NBLK,),
        in_specs=[
            pl.BlockSpec((NBLK, C), lambda i: (i, 0)),
            pl.BlockSpec((NC, NBLK, C), lambda i: (0, i, 0)),
            pl.BlockSpec((C, C), lambda i: (0, 0)),
            pl.BlockSpec((1, C), lambda i: (0, 0)),
            pl.BlockSpec((C, C), lambda i: (0, 0)),
            pl.BlockSpec((1, C), lambda i: (0, 0)),
        ],
        out_specs=pl.BlockSpec((NBLK, C), lambda i: (i, 0)),
        out_shape=jax.ShapeDtypeStruct((N, C), jnp.float32),
    )(h, mout, w1, b1, w2, b2)


def _node_final_body(h_ref, m_ref, w1_ref, b1_ref, w2_ref, b2_ref,
                     hw1_ref, hb1_ref, hw2_ref, hb2_ref, out_ref):
    m = m_ref[0] + m_ref[1]
    t = _ssp(_dot(m, w1_ref[...]) + b1_ref[...])
    h = h_ref[...] + _dot(t, w2_ref[...]) + b2_ref[...]
    t2 = _ssp(_dot(h, hw1_ref[...]) + hb1_ref[...])
    hv = _dot(t2, hw2_ref[...]) + hb2_ref[...]
    part = jnp.sum(hv)

    @pl.when(pl.program_id(0) == 0)
    def _():
        out_ref[...] = jnp.zeros_like(out_ref)

    out_ref[...] += part[None, None]


def _node_final(h, mout, w1, b1, w2, b2, hw1, hb1, hw2, hb2):
    return pl.pallas_call(
        _node_final_body,
        grid=(N // NBLK,),
        in_specs=[
            pl.BlockSpec((NBLK, C), lambda i: (i, 0)),
            pl.BlockSpec((NC, NBLK, C), lambda i: (0, i, 0)),
            pl.BlockSpec((C, C), lambda i: (0, 0)),
            pl.BlockSpec((1, C), lambda i: (0, 0)),
            pl.BlockSpec((C, C), lambda i: (0, 0)),
            pl.BlockSpec((1, C), lambda i: (0, 0)),
            pl.BlockSpec((C, C // 2), lambda i: (0, 0)),
            pl.BlockSpec((1, C // 2), lambda i: (0, 0)),
            pl.BlockSpec((C // 2, 1), lambda i: (0, 0)),
            pl.BlockSpec((1, 1), lambda i: (0, 0)),
        ],
        out_specs=pl.BlockSpec((1, 1), lambda i: (0, 0)),
        out_shape=jax.ShapeDtypeStruct((1, 1), jnp.float32),
    )(h, mout, w1, b1, w2, b2, hw1, hb1, hw2, hb2)


@functools.cache
def _sc_mesh():
    return plsc.VectorSubcoreMesh(core_axis_name="c", subcore_axis_name="s",
                                  num_cores=NC, num_subcores=NS)


def _gather_body(h_hbm, eidx_hbm, hs_out, hd_out,
                 hsp, idx, rows0, rows1, semg0, semg1, semw0, semw1):
    cid = lax.axis_index("c")
    sid = lax.axis_index("s")
    wid = sid * NC + cid

    cpi0 = pltpu.async_copy(eidx_hbm.at[0, pl.ds(wid * IRPW, IRPW)],
                            idx.at[pl.ds(0, IRPW)], semg0)
    cpi1 = pltpu.async_copy(eidx_hbm.at[1, pl.ds(wid * IRPW, IRPW)],
                            idx.at[pl.ds(IRPW, IRPW)], semg1)
    cph = pltpu.async_copy(h_hbm.at[pl.ds(sid * NPS, NPS)],
                           hsp.at[pl.ds(sid * NPS, NPS)], semw0)
    cpi0.wait()
    cpi1.wait()
    cph.wait()
    plsc.subcore_barrier()

    rows = [rows0, rows1]
    semg = [semg0, semg1]
    semw = [semw0, semw1]
    outs = [hs_out, hd_out]
    steps = 2 * NCHUNK

    def issue_gathers(s, buf, sem):
        t, i = divmod(s, NCHUNK)
        cps = []
        for j in range(KJ):
            cps.append(pltpu.async_copy(
                hsp.at[idx.at[t * IRPW + i * KJ + j]],
                buf.at[pl.ds(j * 128, 128)], sem))
        return cps

    def issue_writeout(s, buf, sem):
        t, i = divmod(s, NCHUNK)
        return pltpu.async_copy(buf, outs[t].at[pl.ds(wid * EPW + i * CH, CH)],
                                sem)

    wc = [None, None]
    g = [issue_gathers(0, rows0, semg0), None]
    for s in range(steps):
        b = s % 2
        nb = 1 - b
        if s + 1 < steps:
            if wc[nb] is not None:
                wc[nb].wait()
            g[nb] = issue_gathers(s + 1, rows[nb], semg[nb])
        for cp in g[b]:
            cp.wait()
        wc[b] = issue_writeout(s, rows[b], semw[b])
    wc[0].wait()
    wc[1].wait()


def _gather(hpad, eidx3d):
    return pl.kernel(
        _gather_body,
        out_type=[
            jax.ShapeDtypeStruct((EP, C), jnp.float32),
            jax.ShapeDtypeStruct((EP, C), jnp.float32),
        ],
        mesh=_sc_mesh(),
        compiler_params=pltpu.CompilerParams(use_tc_tiling_on_sc=False),
        scratch_types=[
            pltpu.VMEM_SHARED((NP, C), jnp.float32),
            pltpu.VMEM((2 * IRPW, 128), jnp.int32),
            pltpu.VMEM((CH, C), jnp.float32),
            pltpu.VMEM((CH, C), jnp.float32),
            pltpu.SemaphoreType.DMA,
            pltpu.SemaphoreType.DMA,
            pltpu.SemaphoreType.DMA,
            pltpu.SemaphoreType.DMA,
        ],
    )(hpad, eidx3d)


def _scatter_body(msg_hbm, eidx_hbm, zero_hbm, mout_hbm, macc, mbuf0, mbuf1,
                  idx, semm0, semm1, sema0, sema1):
    cid = lax.axis_index("c")
    sid = lax.axis_index("s")
    wid = sid * NC + cid

    cpi = pltpu.async_copy(eidx_hbm.at[1, pl.ds(wid * IRPW, IRPW)], idx, semm0)
    cpm0 = pltpu.async_copy(msg_hbm.at[pl.ds(wid * EPW, CH)], mbuf0, semm1)
    pltpu.sync_copy(zero_hbm.at[pl.ds(sid * NPS, NPS)],
                    macc.at[pl.ds(sid * NPS, NPS)])
    cpi.wait()
    plsc.subcore_barrier()

    mbuf = [mbuf0, mbuf1]
    semm = [semm0, semm1]
    sema = [sema0, sema1]

    def issue_adds(i, buf, sem):
        cps = []
        for j in range(KJ):
            cps.append(pltpu.async_copy(
                buf.at[pl.ds(j * 128, 128)],
                macc.at[idx.at[i * KJ + j]], sem, add=True))
        return cps

    mc = [cpm0, None]
    ac = [None, None]
    for i in range(NCHUNK):
        b = i % 2
        nb = 1 - b
        if i + 1 < NCHUNK:
            if ac[nb] is not None:
                for cp in ac[nb]:
                    cp.wait()
                ac[nb] = None
            mc[nb] = pltpu.async_copy(
                msg_hbm.at[pl.ds(wid * EPW + (i + 1) * CH, CH)],
                mbuf[nb], semm[nb])
        mc[b].wait()
        ac[b] = issue_adds(i, mbuf[b], sema[b])
    for cps in ac:
        if cps is not None:
            for cp in cps:
                cp.wait()
    plsc.subcore_barrier()
    pltpu.sync_copy(macc.at[pl.ds(sid * NPS, NPS)],
                    mout_hbm.at[cid, pl.ds(sid * NPS, NPS)])


def _scatter(msg, eidx3d, zeros_np):
    return pl.kernel(
        _scatter_body,
        out_type=jax.ShapeDtypeStruct((NC, NP, C), jnp.float32),
        mesh=_sc_mesh(),
        compiler_params=pltpu.CompilerParams(use_tc_tiling_on_sc=False),
        scratch_types=[
            pltpu.VMEM_SHARED((NP, C), jnp.float32),
            pltpu.VMEM((CH, C), jnp.float32),
            pltpu.VMEM((CH, C), jnp.float32),
            pltpu.VMEM((IRPW, 128), jnp.int32),
            pltpu.SemaphoreType.DMA,
            pltpu.SemaphoreType.DMA,
            pltpu.SemaphoreType.DMA,
            pltpu.SemaphoreType.DMA,
        ],
    )(msg, eidx3d, zeros_np)


def _bd(w):
    fin, fout = w.shape
    return (jnp.zeros((2 * fin, 2 * fout), jnp.float32)
            .at[:fin, :fout].set(w).at[fin:, fout:].set(w))


def kernel(z, edge_index, distances, params):
    eidx3d = jnp.pad(edge_index.astype(jnp.int32),
                     ((0, 0), (0, EP - E))).reshape(2, EP // 128, 128)
    dpad = jnp.pad(distances.astype(jnp.float32), (0, EP - E))
    d_even = lax.slice(dpad, (0,), (EP - 1,), (2,))
    d_odd = lax.slice(dpad, (1,), (EP,), (2,))
    zeros_np = jnp.zeros((NP, C), jnp.float32)

    h = _embed(z.reshape(N, 1).astype(jnp.int32), params["emb"])

    ea = d_even
    out = None
    for i in range(3):
        eu = params["eu"][i]
        it = params["it"][i]
        w1 = eu["W1"]
        w1s = _bd(w1[:C])
        w1d = _bd(w1[C:2 * C])
        if i == 0:
            w1e = jnp.zeros((GP, 2 * C), jnp.float32).at[:G].set(w1[2 * C:])
        else:
            w1e = _bd(w1[2 * C:])
        b1 = jnp.tile(eu["b1"], 2).reshape(1, 4 * C)
        w2 = _bd(eu["W2"])
        b2 = jnp.tile(eu["b2"], 2).reshape(1, 2 * C)
        fw = _bd(it["fW"])
        fb = jnp.tile(it["fb"], 2).reshape(1, 2 * C)

        hs, hd = _gather(jnp.pad(h, ((0, NP - N), (0, 0))), eidx3d)
        ea, msg2 = _edge_mlp(i == 0, hs.reshape(EP2, 2 * C),
                             hd.reshape(EP2, 2 * C), ea, d_odd,
                             w1s, w1d, w1e, b1, w2, b2, fw, fb)
        mout = _scatter(msg2.reshape(EP, C), eidx3d, zeros_np)

        m1w1 = it["m1W1"]
        m1b1 = it["m1b1"].reshape(1, C)
        m1w2 = it["m1W2"]
        m1b2 = it["m1b2"].reshape(1, C)
        if i < 2:
            h = _node(h, mout, m1w1, m1b1, m1w2, m1b2)
        else:
            hd_p = params["head"]
            out = _node_final(h, mout, m1w1, m1b1, m1w2, m1b2,
                              hd_p["W1"], hd_p["b1"].reshape(1, C // 2),
                              hd_p["W2"], hd_p["b2"].reshape(1, 1))
    return out


# speedup vs baseline: 6.8606x; 1.0299x over previous
import functools

import jax
import jax.numpy as jnp
from jax import lax
from jax.experimental import pallas as pl
from jax.experimental.pallas import tpu as pltpu
from jax.experimental.pallas import tpu_sc as plsc

N = 10000
E = 160000
C = 64
G = 150

NC = 2
NS = 16
NW = NC * NS
EP = 163840
EPW = EP // NW
CH = 512
KJ = CH // 128
NCHUNK = EPW // CH
IRPW = EPW // 128
NP = 10240
NPS = NP // NS

EBLK = 8192
EP2 = EP // 2
EBLK2 = EBLK // 2
NBLK = 2000
GP = 256

_LOG2 = 0.6931471805599453
_GCOEF = 0.1 ** 0.5


def _ssp(x):
    return jnp.maximum(x, 0.0) + jnp.log1p(jnp.exp(-jnp.abs(x))) - _LOG2


def _dot(a, b):
    return jnp.dot(a, b, preferred_element_type=jnp.float32)


def _embed_body(z_ref, emb_ref, h_ref):
    zb = z_ref[...]
    oh = (zb == lax.broadcasted_iota(jnp.int32, (NBLK, 100), 1)).astype(jnp.float32)
    h_ref[...] = _dot(oh, emb_ref[...])


def _embed(z2d, emb):
    return pl.pallas_call(
        _embed_body,
        grid=(N // NBLK,),
        in_specs=[
            pl.BlockSpec((NBLK, 1), lambda i: (i, 0)),
            pl.BlockSpec((100, C), lambda i: (0, 0)),
        ],
        out_specs=pl.BlockSpec((NBLK, C), lambda i: (i, 0)),
        out_shape=jax.ShapeDtypeStruct((N, C), jnp.float32),
    )(z2d, emb)


def _edge_mlp_body(first, hs_ref, hd_ref, dea_ref, dob_ref, w1s_ref, w1d_ref,
                   w1e_ref, b1_ref, w2_ref, b2_ref, fw_ref, fb_ref,
                   ea_out_ref, msg_ref):
    hs = hs_ref[...]
    hd = hd_ref[...]
    if first:
        de = dea_ref[...].reshape(EBLK2, 1)
        do = dob_ref[...].reshape(EBLK2, 1)
        kk = lax.broadcasted_iota(jnp.int32, (EBLK2, GP), 1).astype(jnp.float32)
        dfe = de - 0.1 * kk
        dfo = do - 0.1 * kk
        ge = jnp.exp(-_GCOEF * dfe * dfe)
        go = jnp.exp(-_GCOEF * dfo * dfo)
        pre_e = jnp.concatenate([_dot(ge, w1e_ref[...]),
                                 _dot(go, w1e_ref[...])], axis=1)
    else:
        pre_e = _dot(dea_ref[...], w1e_ref[...])
    pre = (_dot(hs, w1s_ref[...]) + _dot(hd, w1d_ref[...])
           + pre_e + b1_ref[...])
    ea = _dot(_ssp(pre), w2_ref[...]) + b2_ref[...]
    hh = _dot(hs, fw_ref[...]) + fb_ref[...]
    msg = hh * ea
    pid = pl.program_id(0)
    nfull = EP // EBLK - 2

    @pl.when(pid < nfull)
    def _():
        ea_out_ref[...] = ea
        msg_ref[...] = msg

    @pl.when(pid >= nfull)
    def _():
        rid = pid * EBLK2 + lax.broadcasted_iota(jnp.int32, (EBLK2, 1), 0)
        valid = (rid < E // 2).astype(jnp.float32)
        ea_out_ref[...] = ea * valid
        msg_ref[...] = msg * valid


def _edge_mlp(first, hs2, hd2, ea_or_de, dodd, w1s, w1d, w1e, b1, w2, b2,
              fw, fb):
    if first:
        ea_spec = pl.BlockSpec((EBLK2,), lambda i: (i,))
        do_spec = pl.BlockSpec((EBLK2,), lambda i: (i,))
    else:
        ea_spec = pl.BlockSpec((EBLK2, 2 * C), lambda i: (i, 0))
        do_spec = pl.BlockSpec((EBLK2,), lambda i: (0,))
    win = w1e.shape[0]
    wout = w1e.shape[1]
    return pl.pallas_call(
        functools.partial(_edge_mlp_body, first),
        grid=(EP // EBLK,),
        in_specs=[
            pl.BlockSpec((EBLK2, 2 * C), lambda i: (i, 0)),
            pl.BlockSpec((EBLK2, 2 * C), lambda i: (i, 0)),
            ea_spec,
            do_spec,
            pl.BlockSpec((2 * C, 4 * C), lambda i: (0, 0)),
            pl.BlockSpec((2 * C, 4 * C), lambda i: (0, 0)),
            pl.BlockSpec((win, wout), lambda i: (0, 0)),
            pl.BlockSpec((1, 4 * C), lambda i: (0, 0)),
            pl.BlockSpec((4 * C, 2 * C), lambda i: (0, 0)),
            pl.BlockSpec((1, 2 * C), lambda i: (0, 0)),
            pl.BlockSpec((2 * C, 2 * C), lambda i: (0, 0)),
            pl.BlockSpec((1, 2 * C), lambda i: (0, 0)),
        ],
        out_specs=[
            pl.BlockSpec((EBLK2, 2 * C), lambda i: (i, 0)),
            pl.BlockSpec((EBLK2, 2 * C), lambda i: (i, 0)),
        ],
        out_shape=[
            jax.ShapeDtypeStruct((EP2, 2 * C), jnp.float32),
            jax.ShapeDtypeStruct((EP2, 2 * C), jnp.float32),
        ],
    )(hs2, hd2, ea_or_de, dodd, w1s, w1d, w1e, b1, w2, b2, fw, fb)


def _node_body(h_ref, m_ref, w1_ref, b1_ref, w2_ref, b2_ref, h_out_ref):
    m = m_ref[0] + m_ref[1]
    t = _ssp(_dot(m, w1_ref[...]) + b1_ref[...])
    h_out_ref[...] = h_ref[...] + _dot(t, w2_ref[...]) + b2_ref[...]


def _node(h, mout, w1, b1, w2, b2):
    return pl.pallas_call(
        _node_body,
        grid=(N // NBLK,),
        in_specs=[
            pl.BlockSpec((NBLK, C), lambda i: (i, 0)),
            pl.BlockSpec((NC, NBLK, C), lambda i: (0, i, 0)),
            pl.BlockSpec((C, C), lambda i: (0, 0)),
            pl.BlockSpec((1, C), lambda i: (0, 0)),
            pl.BlockSpec((C, C), lambda i: (0, 0)),
            pl.BlockSpec((1, C), lambda i: (0, 0)),
        ],
        out_specs=pl.BlockSpec((NBLK, C), lambda i: (i, 0)),
        out_shape=jax.ShapeDtypeStruct((N, C), jnp.float32),
    )(h, mout, w1, b1, w2, b2)


def _node_final_body(h_ref, m_ref, w1_ref, b1_ref, w2_ref, b2_ref,
                     hw1_ref, hb1_ref, hw2_ref, hb2_ref, out_ref):
    m = m_ref[0] + m_ref[1]
    t = _ssp(_dot(m, w1_ref[...]) + b1_ref[...])
    h = h_ref[...] + _dot(t, w2_ref[...]) + b2_ref[...]
    t2 = _ssp(_dot(h, hw1_ref[...]) + hb1_ref[...])
    hv = _dot(t2, hw2_ref[...]) + hb2_ref[...]
    part = jnp.sum(hv)

    @pl.when(pl.program_id(0) == 0)
    def _():
        out_ref[...] = jnp.zeros_like(out_ref)

    out_ref[...] += part[None, None]


def _node_final(h, mout, w1, b1, w2, b2, hw1, hb1, hw2, hb2):
    return pl.pallas_call(
        _node_final_body,
        grid=(N // NBLK,),
        in_specs=[
            pl.BlockSpec((NBLK, C), lambda i: (i, 0)),
            pl.BlockSpec((NC, NBLK, C), lambda i: (0, i, 0)),
            pl.BlockSpec((C, C), lambda i: (0, 0)),
            pl.BlockSpec((1, C), lambda i: (0, 0)),
            pl.BlockSpec((C, C), lambda i: (0, 0)),
            pl.BlockSpec((1, C), lambda i: (0, 0)),
            pl.BlockSpec((C, C // 2), lambda i: (0, 0)),
            pl.BlockSpec((1, C // 2), lambda i: (0, 0)),
            pl.BlockSpec((C // 2, 1), lambda i: (0, 0)),
            pl.BlockSpec((1, 1), lambda i: (0, 0)),
        ],
        out_specs=pl.BlockSpec((1, 1), lambda i: (0, 0)),
        out_shape=jax.ShapeDtypeStruct((1, 1), jnp.float32),
    )(h, mout, w1, b1, w2, b2, hw1, hb1, hw2, hb2)


@functools.cache
def _sc_mesh():
    return plsc.VectorSubcoreMesh(core_axis_name="c", subcore_axis_name="s",
                                  num_cores=NC, num_subcores=NS)


def _gather_body(h_hbm, eidx_hbm, hs_out, hd_out,
                 hsp, idx, rows0, rows1, semg0, semg1, semw0, semw1):
    cid = lax.axis_index("c")
    sid = lax.axis_index("s")
    wid = sid * NC + cid

    cpi0 = pltpu.async_copy(eidx_hbm.at[0, pl.ds(wid * IRPW, IRPW)],
                            idx.at[pl.ds(0, IRPW)], semg0)
    cpi1 = pltpu.async_copy(eidx_hbm.at[1, pl.ds(wid * IRPW, IRPW)],
                            idx.at[pl.ds(IRPW, IRPW)], semg1)
    cph = pltpu.async_copy(h_hbm.at[pl.ds(sid * NPS, NPS)],
                           hsp.at[pl.ds(sid * NPS, NPS)], semw0)
    cpi0.wait()
    cpi1.wait()
    cph.wait()
    plsc.subcore_barrier()

    rows = [rows0, rows1]
    semg = [semg0, semg1]
    semw = [semw0, semw1]
    outs = [hs_out, hd_out]
    steps = 2 * NCHUNK

    def issue_gathers(s, buf, sem):
        t, i = divmod(s, NCHUNK)
        cps = []
        for j in range(KJ):
            cps.append(pltpu.async_copy(
                hsp.at[idx.at[t * IRPW + i * KJ + j]],
                buf.at[pl.ds(j * 128, 128)], sem))
        return cps

    def issue_writeout(s, buf, sem):
        t, i = divmod(s, NCHUNK)
        return pltpu.async_copy(buf, outs[t].at[pl.ds(wid * EPW + i * CH, CH)],
                                sem)

    wc = [None, None]
    g = [issue_gathers(0, rows0, semg0), None]
    for s in range(steps):
        b = s % 2
        nb = 1 - b
        if s + 1 < steps:
            if wc[nb] is not None:
                wc[nb].wait()
            g[nb] = issue_gathers(s + 1, rows[nb], semg[nb])
        for cp in g[b]:
            cp.wait()
        wc[b] = issue_writeout(s, rows[b], semw[b])
    wc[0].wait()
    wc[1].wait()


def _gather(hpad, eidx3d):
    return pl.kernel(
        _gather_body,
        out_type=[
            jax.ShapeDtypeStruct((EP, C), jnp.float32),
            jax.ShapeDtypeStruct((EP, C), jnp.float32),
        ],
        mesh=_sc_mesh(),
        compiler_params=pltpu.CompilerParams(use_tc_tiling_on_sc=False),
        scratch_types=[
            pltpu.VMEM_SHARED((NP, C), jnp.float32),
            pltpu.VMEM((2 * IRPW, 128), jnp.int32),
            pltpu.VMEM((CH, C), jnp.float32),
            pltpu.VMEM((CH, C), jnp.float32),
            pltpu.SemaphoreType.DMA,
            pltpu.SemaphoreType.DMA,
            pltpu.SemaphoreType.DMA,
            pltpu.SemaphoreType.DMA,
        ],
    )(hpad, eidx3d)


def _scatter_body(msg_hbm, eidx_hbm, zero_hbm, mout_hbm, macc, mbuf0, mbuf1,
                  idx, semm0, semm1, sema0, sema1):
    cid = lax.axis_index("c")
    sid = lax.axis_index("s")
    wid = sid * NC + cid

    cpi = pltpu.async_copy(eidx_hbm.at[1, pl.ds(wid * IRPW, IRPW)], idx, semm0)
    cpm0 = pltpu.async_copy(msg_hbm.at[pl.ds(wid * EPW, CH)], mbuf0, semm1)
    pltpu.sync_copy(zero_hbm.at[pl.ds(sid * NPS, NPS)],
                    macc.at[pl.ds(sid * NPS, NPS)])
    cpi.wait()
    plsc.subcore_barrier()

    mbuf = [mbuf0, mbuf1]
    semm = [semm0, semm1]
    sema = [sema0, sema1]

    def issue_adds(i, buf, sem):
        cps = []
        for j in range(KJ):
            cps.append(pltpu.async_copy(
                buf.at[pl.ds(j * 128, 128)],
                macc.at[idx.at[i * KJ + j]], sem, add=True))
        return cps

    mc = [cpm0, None]
    ac = [None, None]
    for i in range(NCHUNK):
        b = i % 2
        nb = 1 - b
        if i + 1 < NCHUNK:
            if ac[nb] is not None:
                for cp in ac[nb]:
                    cp.wait()
                ac[nb] = None
            mc[nb] = pltpu.async_copy(
                msg_hbm.at[pl.ds(wid * EPW + (i + 1) * CH, CH)],
                mbuf[nb], semm[nb])
        mc[b].wait()
        ac[b] = issue_adds(i, mbuf[b], sema[b])
    for cps in ac:
        if cps is not None:
            for cp in cps:
                cp.wait()
    plsc.subcore_barrier()
    pltpu.sync_copy(macc.at[pl.ds(sid * NPS, NPS)],
                    mout_hbm.at[cid, pl.ds(sid * NPS, NPS)])


def _scatter(msg, eidx3d, zeros_np):
    return pl.kernel(
        _scatter_body,
        out_type=jax.ShapeDtypeStruct((NC, NP, C), jnp.float32),
        mesh=_sc_mesh(),
        compiler_params=pltpu.CompilerParams(use_tc_tiling_on_sc=False),
        scratch_types=[
            pltpu.VMEM_SHARED((NP, C), jnp.float32),
            pltpu.VMEM((CH, C), jnp.float32),
            pltpu.VMEM((CH, C), jnp.float32),
            pltpu.VMEM((IRPW, 128), jnp.int32),
            pltpu.SemaphoreType.DMA,
            pltpu.SemaphoreType.DMA,
            pltpu.SemaphoreType.DMA,
            pltpu.SemaphoreType.DMA,
        ],
    )(msg, eidx3d, zeros_np)


def _bd(w):
    fin, fout = w.shape
    return (jnp.zeros((2 * fin, 2 * fout), jnp.float32)
            .at[:fin, :fout].set(w).at[fin:, fout:].set(w))


def kernel(z, edge_index, distances, params):
    eidx3d = jnp.pad(edge_index.astype(jnp.int32),
                     ((0, 0), (0, EP - E))).reshape(2, EP // 128, 128)
    dpad = jnp.pad(distances.astype(jnp.float32), (0, EP - E))
    d_even = lax.slice(dpad, (0,), (EP - 1,), (2,))
    d_odd = lax.slice(dpad, (1,), (EP,), (2,))
    zeros_np = jnp.zeros((NP, C), jnp.float32)

    h = _embed(z.reshape(N, 1).astype(jnp.int32), params["emb"])

    ea = d_even
    out = None
    for i in range(3):
        eu = params["eu"][i]
        it = params["it"][i]
        w1 = eu["W1"]
        w1s = _bd(w1[:C])
        w1d = _bd(w1[C:2 * C])
        if i == 0:
            w1e = jnp.zeros((GP, 2 * C), jnp.float32).at[:G].set(w1[2 * C:])
        else:
            w1e = _bd(w1[2 * C:])
        b1 = jnp.tile(eu["b1"], 2).reshape(1, 4 * C)
        w2 = _bd(eu["W2"])
        b2 = jnp.tile(eu["b2"], 2).reshape(1, 2 * C)
        fw = _bd(it["fW"])
        fb = jnp.tile(it["fb"], 2).reshape(1, 2 * C)

        hs, hd = _gather(jnp.pad(h, ((0, NP - N), (0, 0))), eidx3d)
        ea, msg2 = _edge_mlp(i == 0, hs.reshape(EP2, 2 * C),
                             hd.reshape(EP2, 2 * C), ea, d_odd,
                             w1s, w1d, w1e, b1, w2, b2, fw, fb)
        mout = _scatter(msg2.reshape(EP, C), eidx3d, zeros_np)

        m1w1 = it["m1W1"]
        m1b1 = it["m1b1"].reshape(1, C)
        m1w2 = it["m1W2"]
        m1b2 = it["m1b2"].reshape(1, C)
        if i < 2:
            h = _node(h, mout, m1w1, m1b1, m1w2, m1b2)
        else:
            hd_p = params["head"]
            out = _node_final(h, mout, m1w1, m1b1, m1w2, m1b2,
                              hd_p["W1"], hd_p["b1"].reshape(1, C // 2),
                              hd_p["W2"], hd_p["b2"].reshape(1, 1))
    return out


# speedup vs baseline: 7.3137x; 1.0661x over previous
import functools

import jax
import jax.numpy as jnp
from jax import lax
from jax.experimental import pallas as pl
from jax.experimental.pallas import tpu as pltpu
from jax.experimental.pallas import tpu_sc as plsc

N = 10000
E = 160000
C = 64
G = 150

NC = 2
NS = 16
NW = NC * NS
EP = 163840
EPW = EP // NW
CH = 512
KJ = CH // 128
NCHUNK = EPW // CH
IRPW = EPW // 128
NP = 10240
NPS = NP // NS

EBLK = 8192
EP2 = EP // 2
EBLK2 = EBLK // 2

EP_H = EP // 2
EP2_H = EP_H // 2
EPW_H = EP_H // NW
NCHUNK_H = EPW_H // CH
IRPW_H = EPW_H // 128
IRH = EP_H // 128
NBLK = 2000
GP = 256

_LOG2 = 0.6931471805599453
_GCOEF = 0.1 ** 0.5


def _ssp(x):
    return jnp.maximum(x, 0.0) + jnp.log1p(jnp.exp(-jnp.abs(x))) - _LOG2


def _dot(a, b):
    return jnp.dot(a, b, preferred_element_type=jnp.float32)


def _embed_body(z_ref, emb_ref, h_ref):
    zb = z_ref[...]
    oh = (zb == lax.broadcasted_iota(jnp.int32, (NBLK, 100), 1)).astype(jnp.float32)
    h_ref[...] = _dot(oh, emb_ref[...])


def _embed(z2d, emb):
    return pl.pallas_call(
        _embed_body,
        grid=(N // NBLK,),
        in_specs=[
            pl.BlockSpec((NBLK, 1), lambda i: (i, 0)),
            pl.BlockSpec((100, C), lambda i: (0, 0)),
        ],
        out_specs=pl.BlockSpec((NBLK, C), lambda i: (i, 0)),
        out_shape=jax.ShapeDtypeStruct((N, C), jnp.float32),
    )(z2d, emb)


def _edge_mlp_body(first, half, hs_ref, hd_ref, dea_ref, dob_ref, w1s_ref, w1d_ref,
                   w1e_ref, b1_ref, w2_ref, b2_ref, fw_ref, fb_ref,
                   ea_out_ref, msg_ref):
    hs = hs_ref[...]
    hd = hd_ref[...]
    if first:
        de = dea_ref[...].reshape(EBLK2, 1)
        do = dob_ref[...].reshape(EBLK2, 1)
        kk = lax.broadcasted_iota(jnp.int32, (EBLK2, GP), 1).astype(jnp.float32)
        dfe = de - 0.1 * kk
        dfo = do - 0.1 * kk
        ge = jnp.exp(-_GCOEF * dfe * dfe)
        go = jnp.exp(-_GCOEF * dfo * dfo)
        pre_e = jnp.concatenate([_dot(ge, w1e_ref[...]),
                                 _dot(go, w1e_ref[...])], axis=1)
    else:
        pre_e = _dot(dea_ref[...], w1e_ref[...])
    pre = (_dot(hs, w1s_ref[...]) + _dot(hd, w1d_ref[...])
           + pre_e + b1_ref[...])
    ea = _dot(_ssp(pre), w2_ref[...]) + b2_ref[...]
    hh = _dot(hs, fw_ref[...]) + fb_ref[...]
    msg = hh * ea
    pid = pl.program_id(0)
    nfull = (E // 2 - half * EP2_H) // EBLK2

    @pl.when(pid < nfull)
    def _():
        ea_out_ref[...] = ea
        msg_ref[...] = msg

    @pl.when(pid >= nfull)
    def _():
        rid = (half * EP2_H + pid * EBLK2
               + lax.broadcasted_iota(jnp.int32, (EBLK2, 1), 0))
        valid = (rid < E // 2).astype(jnp.float32)
        ea_out_ref[...] = ea * valid
        msg_ref[...] = msg * valid


def _edge_mlp(first, half, hs2, hd2, ea_or_de, dodd, w1s, w1d, w1e, b1, w2,
              b2, fw, fb):
    if first:
        ea_spec = pl.BlockSpec((EBLK2,), lambda i: (i,))
        do_spec = pl.BlockSpec((EBLK2,), lambda i: (i,))
    else:
        ea_spec = pl.BlockSpec((EBLK2, 2 * C), lambda i: (i, 0))
        do_spec = pl.BlockSpec((EBLK2,), lambda i: (0,))
    win = w1e.shape[0]
    wout = w1e.shape[1]
    return pl.pallas_call(
        functools.partial(_edge_mlp_body, first, half),
        grid=(EP_H // EBLK,),
        in_specs=[
            pl.BlockSpec((EBLK2, 2 * C), lambda i: (i, 0)),
            pl.BlockSpec((EBLK2, 2 * C), lambda i: (i, 0)),
            ea_spec,
            do_spec,
            pl.BlockSpec((2 * C, 4 * C), lambda i: (0, 0)),
            pl.BlockSpec((2 * C, 4 * C), lambda i: (0, 0)),
            pl.BlockSpec((win, wout), lambda i: (0, 0)),
            pl.BlockSpec((1, 4 * C), lambda i: (0, 0)),
            pl.BlockSpec((4 * C, 2 * C), lambda i: (0, 0)),
            pl.BlockSpec((1, 2 * C), lambda i: (0, 0)),
            pl.BlockSpec((2 * C, 2 * C), lambda i: (0, 0)),
            pl.BlockSpec((1, 2 * C), lambda i: (0, 0)),
        ],
        out_specs=[
            pl.BlockSpec((EBLK2, 2 * C), lambda i: (i, 0)),
            pl.BlockSpec((EBLK2, 2 * C), lambda i: (i, 0)),
        ],
        out_shape=[
            jax.ShapeDtypeStruct((EP2_H, 2 * C), jnp.float32),
            jax.ShapeDtypeStruct((EP2_H, 2 * C), jnp.float32),
        ],
    )(hs2, hd2, ea_or_de, dodd, w1s, w1d, w1e, b1, w2, b2, fw, fb)


def _node_body(h_ref, m_ref, m2_ref, w1_ref, b1_ref, w2_ref, b2_ref,
               h_out_ref):
    m = m_ref[0] + m_ref[1] + m2_ref[0] + m2_ref[1]
    t = _ssp(_dot(m, w1_ref[...]) + b1_ref[...])
    h_out_ref[...] = h_ref[...] + _dot(t, w2_ref[...]) + b2_ref[...]


def _node(h, mout, mout2, w1, b1, w2, b2):
    return pl.pallas_call(
        _node_body,
        grid=(N // NBLK,),
        in_specs=[
            pl.BlockSpec((NBLK, C), lambda i: (i, 0)),
            pl.BlockSpec((NC, NBLK, C), lambda i: (0, i, 0)),
            pl.BlockSpec((NC, NBLK, C), lambda i: (0, i, 0)),
            pl.BlockSpec((C, C), lambda i: (0, 0)),
            pl.BlockSpec((1, C), lambda i: (0, 0)),
            pl.BlockSpec((C, C), lambda i: (0, 0)),
            pl.BlockSpec((1, C), lambda i: (0, 0)),
        ],
        out_specs=pl.BlockSpec((NBLK, C), lambda i: (i, 0)),
        out_shape=jax.ShapeDtypeStruct((N, C), jnp.float32),
    )(h, mout, mout2, w1, b1, w2, b2)


def _node_final_body(h_ref, m_ref, m2_ref, w1_ref, b1_ref, w2_ref, b2_ref,
                     hw1_ref, hb1_ref, hw2_ref, hb2_ref, out_ref):
    m = m_ref[0] + m_ref[1] + m2_ref[0] + m2_ref[1]
    t = _ssp(_dot(m, w1_ref[...]) + b1_ref[...])
    h = h_ref[...] + _dot(t, w2_ref[...]) + b2_ref[...]
    t2 = _ssp(_dot(h, hw1_ref[...]) + hb1_ref[...])
    hv = _dot(t2, hw2_ref[...]) + hb2_ref[...]
    part = jnp.sum(hv)

    @pl.when(pl.program_id(0) == 0)
    def _():
        out_ref[...] = jnp.zeros_like(out_ref)

    out_ref[...] += part[None, None]


def _node_final(h, mout, mout2, w1, b1, w2, b2, hw1, hb1, hw2, hb2):
    return pl.pallas_call(
        _node_final_body,
        grid=(N // NBLK,),
        in_specs=[
            pl.BlockSpec((NBLK, C), lambda i: (i, 0)),
            pl.BlockSpec((NC, NBLK, C), lambda i: (0, i, 0)),
            pl.BlockSpec((NC, NBLK, C), lambda i: (0, i, 0)),
            pl.BlockSpec((C, C), lambda i: (0, 0)),
            pl.BlockSpec((1, C), lambda i: (0, 0)),
            pl.BlockSpec((C, C), lambda i: (0, 0)),
            pl.BlockSpec((1, C), lambda i: (0, 0)),
            pl.BlockSpec((C, C // 2), lambda i: (0, 0)),
            pl.BlockSpec((1, C // 2), lambda i: (0, 0)),
            pl.BlockSpec((C // 2, 1), lambda i: (0, 0)),
            pl.BlockSpec((1, 1), lambda i: (0, 0)),
        ],
        out_specs=pl.BlockSpec((1, 1), lambda i: (0, 0)),
        out_shape=jax.ShapeDtypeStruct((1, 1), jnp.float32),
    )(h, mout, mout2, w1, b1, w2, b2, hw1, hb1, hw2, hb2)


@functools.cache
def _sc_mesh():
    return plsc.VectorSubcoreMesh(core_axis_name="c", subcore_axis_name="s",
                                  num_cores=NC, num_subcores=NS)


def _gather_body(half, h_hbm, eidx_hbm, hs_out, hd_out,
                 hsp, idx, rows0, rows1, semg0, semg1, semw0, semw1):
    cid = lax.axis_index("c")
    sid = lax.axis_index("s")
    wid = sid * NC + cid

    r0 = half * IRH + wid * IRPW_H
    cpi0 = pltpu.async_copy(eidx_hbm.at[0, pl.ds(r0, IRPW_H)],
                            idx.at[pl.ds(0, IRPW_H)], semg0)
    cpi1 = pltpu.async_copy(eidx_hbm.at[1, pl.ds(r0, IRPW_H)],
                            idx.at[pl.ds(IRPW_H, IRPW_H)], semg1)
    cph = pltpu.async_copy(h_hbm.at[pl.ds(sid * NPS, NPS)],
                           hsp.at[pl.ds(sid * NPS, NPS)], semw0)
    cpi0.wait()
    cpi1.wait()
    cph.wait()
    plsc.subcore_barrier()

    rows = [rows0, rows1]
    semg = [semg0, semg1]
    semw = [semw0, semw1]
    outs = [hs_out, hd_out]
    steps = 2 * NCHUNK_H

    def issue_gathers(s, buf, sem):
        t, i = divmod(s, NCHUNK_H)
        cps = []
        for j in range(KJ):
            cps.append(pltpu.async_copy(
                hsp.at[idx.at[t * IRPW_H + i * KJ + j]],
                buf.at[pl.ds(j * 128, 128)], sem))
        return cps

    def issue_writeout(s, buf, sem):
        t, i = divmod(s, NCHUNK_H)
        return pltpu.async_copy(buf,
                                outs[t].at[pl.ds(wid * EPW_H + i * CH, CH)],
                                sem)

    wc = [None, None]
    g = [issue_gathers(0, rows0, semg0), None]
    for s in range(steps):
        b = s % 2
        nb = 1 - b
        if s + 1 < steps:
            if wc[nb] is not None:
                wc[nb].wait()
            g[nb] = issue_gathers(s + 1, rows[nb], semg[nb])
        for cp in g[b]:
            cp.wait()
        wc[b] = issue_writeout(s, rows[b], semw[b])
    wc[0].wait()
    wc[1].wait()


def _gather(hpad, eidx3d, half):
    return pl.kernel(
        functools.partial(_gather_body, half),
        out_type=[
            jax.ShapeDtypeStruct((EP_H, C), jnp.float32),
            jax.ShapeDtypeStruct((EP_H, C), jnp.float32),
        ],
        mesh=_sc_mesh(),
        compiler_params=pltpu.CompilerParams(use_tc_tiling_on_sc=False),
        scratch_types=[
            pltpu.VMEM_SHARED((NP, C), jnp.float32),
            pltpu.VMEM((2 * IRPW_H, 128), jnp.int32),
            pltpu.VMEM((CH, C), jnp.float32),
            pltpu.VMEM((CH, C), jnp.float32),
            pltpu.SemaphoreType.DMA,
            pltpu.SemaphoreType.DMA,
            pltpu.SemaphoreType.DMA,
            pltpu.SemaphoreType.DMA,
        ],
    )(hpad, eidx3d)


def _scatter_body(half, msg_hbm, eidx_hbm, zero_hbm, mout_hbm, macc,
                  mbuf0, mbuf1, idx, semm0, semm1, sema0, sema1):
    cid = lax.axis_index("c")
    sid = lax.axis_index("s")
    wid = sid * NC + cid

    cpi = pltpu.async_copy(
        eidx_hbm.at[1, pl.ds(half * IRH + wid * IRPW_H, IRPW_H)], idx, semm0)
    cpm0 = pltpu.async_copy(msg_hbm.at[pl.ds(wid * EPW_H, CH)], mbuf0, semm1)
    pltpu.sync_copy(zero_hbm.at[pl.ds(sid * NPS, NPS)],
                    macc.at[pl.ds(sid * NPS, NPS)])
    cpi.wait()
    plsc.subcore_barrier()

    mbuf = [mbuf0, mbuf1]
    semm = [semm0, semm1]
    sema = [sema0, sema1]

    def issue_adds(i, buf, sem):
        cps = []
        for j in range(KJ):
            cps.append(pltpu.async_copy(
                buf.at[pl.ds(j * 128, 128)],
                macc.at[idx.at[i * KJ + j]], sem, add=True))
        return cps

    mc = [cpm0, None]
    ac = [None, None]
    for i in range(NCHUNK_H):
        b = i % 2
        nb = 1 - b
        if i + 1 < NCHUNK_H:
            if ac[nb] is not None:
                for cp in ac[nb]:
                    cp.wait()
                ac[nb] = None
            mc[nb] = pltpu.async_copy(
                msg_hbm.at[pl.ds(wid * EPW_H + (i + 1) * CH, CH)],
                mbuf[nb], semm[nb])
        mc[b].wait()
        ac[b] = issue_adds(i, mbuf[b], sema[b])
    for cps in ac:
        if cps is not None:
            for cp in cps:
                cp.wait()
    plsc.subcore_barrier()
    pltpu.sync_copy(macc.at[pl.ds(sid * NPS, NPS)],
                    mout_hbm.at[cid, pl.ds(sid * NPS, NPS)])


def _scatter(msg, eidx3d, zeros_np, half):
    return pl.kernel(
        functools.partial(_scatter_body, half),
        out_type=jax.ShapeDtypeStruct((NC, NP, C), jnp.float32),
        mesh=_sc_mesh(),
        compiler_params=pltpu.CompilerParams(use_tc_tiling_on_sc=False),
        scratch_types=[
            pltpu.VMEM_SHARED((NP, C), jnp.float32),
            pltpu.VMEM((CH, C), jnp.float32),
            pltpu.VMEM((CH, C), jnp.float32),
            pltpu.VMEM((IRPW_H, 128), jnp.int32),
            pltpu.SemaphoreType.DMA,
            pltpu.SemaphoreType.DMA,
            pltpu.SemaphoreType.DMA,
            pltpu.SemaphoreType.DMA,
        ],
    )(msg, eidx3d, zeros_np)


def _bd(w):
    fin, fout = w.shape
    return (jnp.zeros((2 * fin, 2 * fout), jnp.float32)
            .at[:fin, :fout].set(w).at[fin:, fout:].set(w))


def kernel(z, edge_index, distances, params):
    eidx3d = jnp.pad(edge_index.astype(jnp.int32),
                     ((0, 0), (0, EP - E))).reshape(2, EP // 128, 128)
    dpad = jnp.pad(distances.astype(jnp.float32), (0, EP - E))
    d_even = lax.slice(dpad, (0,), (EP - 1,), (2,))
    d_odd = lax.slice(dpad, (1,), (EP,), (2,))
    d_even_h = [lax.slice(d_even, (k * EP2_H,), ((k + 1) * EP2_H,))
                for k in range(2)]
    d_odd_h = [lax.slice(d_odd, (k * EP2_H,), ((k + 1) * EP2_H,))
               for k in range(2)]
    zeros_np = jnp.zeros((NP, C), jnp.float32)

    h = _embed(z.reshape(N, 1).astype(jnp.int32), params["emb"])

    ea_h = list(d_even_h)
    out = None
    for i in range(3):
        eu = params["eu"][i]
        it = params["it"][i]
        w1 = eu["W1"]
        w1s = _bd(w1[:C])
        w1d = _bd(w1[C:2 * C])
        if i == 0:
            w1e = jnp.zeros((GP, 2 * C), jnp.float32).at[:G].set(w1[2 * C:])
        else:
            w1e = _bd(w1[2 * C:])
        b1 = jnp.tile(eu["b1"], 2).reshape(1, 4 * C)
        w2 = _bd(eu["W2"])
        b2 = jnp.tile(eu["b2"], 2).reshape(1, 2 * C)
        fw = _bd(it["fW"])
        fb = jnp.tile(it["fb"], 2).reshape(1, 2 * C)

        hpadded = jnp.pad(h, ((0, NP - N), (0, 0)))
        gat = [_gather(hpadded, eidx3d, k) for k in range(2)]
        mouts = []
        for k in range(2):
            hs, hd = gat[k]
            ea_h[k], msg2 = _edge_mlp(i == 0, k, hs.reshape(EP2_H, 2 * C),
                                      hd.reshape(EP2_H, 2 * C), ea_h[k],
                                      d_odd_h[k], w1s, w1d, w1e, b1, w2, b2,
                                      fw, fb)
            mouts.append(_scatter(msg2.reshape(EP_H, C), eidx3d, zeros_np, k))

        m1w1 = it["m1W1"]
        m1b1 = it["m1b1"].reshape(1, C)
        m1w2 = it["m1W2"]
        m1b2 = it["m1b2"].reshape(1, C)
        if i < 2:
            h = _node(h, mouts[0], mouts[1], m1w1, m1b1, m1w2, m1b2)
        else:
            hd_p = params["head"]
            out = _node_final(h, mouts[0], mouts[1], m1w1, m1b1, m1w2, m1b2,
                              hd_p["W1"], hd_p["b1"].reshape(1, C // 2),
                              hd_p["W2"], hd_p["b2"].reshape(1, 1))
    return out
